# Initial kernel scaffold; baseline (speedup 1.0000x reference)
import jax
import jax.numpy as jnp
from jax.experimental import pallas as pl


def kernel(edge_embeddings, edge_index, edge_attr, node_embeddings, num_nodes, attn_W, attn_b, update_W, update_b, edge_lin_W, edge_lin_b, node_lin_W, node_lin_b):
    raise NotImplementedError("write your pallas kernel here")


# speedup vs baseline: 1.4506x; 1.4506x over previous
import dataclasses
import functools

import jax
import jax.numpy as jnp
from jax import lax
from jax.experimental import pallas as pl
from jax.experimental.pallas import tpu as pltpu
from jax.experimental.pallas import tpu_sc as plsc


def _node_body(emb_ref, ws_ref, wd_ref, wl_ref, nb_ref, wa_ref,
               tsrc_ref, tdst_ref, p2_ref):
    emb = emb_ref[...]
    d = emb.shape[1]
    pl_feat = jnp.dot(emb, wl_ref[...], preferred_element_type=jnp.float32)
    pl_feat = pl_feat + nb_ref[...]
    tsrc_ref[:, :d] = jnp.dot(emb, ws_ref[...],
                              preferred_element_type=jnp.float32)
    tsrc_ref[:, d:] = pl_feat
    tdst_ref[:, :d] = jnp.dot(emb, wd_ref[...],
                              preferred_element_type=jnp.float32)
    tdst_ref[:, d:] = pl_feat
    p2_ref[...] = jnp.dot(emb, wa_ref[...], preferred_element_type=jnp.float32)


def _node_precompute(emb, ws, wd, wl, nb, wa, bn):
    n, d = emb.shape
    grid = (n // bn,)
    full = lambda shape: pl.BlockSpec(shape, lambda i: (0, 0))
    return pl.pallas_call(
        _node_body,
        grid=grid,
        in_specs=[
            pl.BlockSpec((bn, d), lambda i: (i, 0)),
            full((d, d)), full((d, d)), full((d, d)),
            full((1, d)), full((d, 2)),
        ],
        out_specs=[
            pl.BlockSpec((bn, 2 * d), lambda i: (i, 0)),
            pl.BlockSpec((bn, 2 * d), lambda i: (i, 0)),
            pl.BlockSpec((bn, 2), lambda i: (i, 0)),
        ],
        out_shape=[
            jax.ShapeDtypeStruct((n, 2 * d), jnp.float32),
            jax.ShapeDtypeStruct((n, 2 * d), jnp.float32),
            jax.ShapeDtypeStruct((n, 2), jnp.float32),
        ],
        compiler_params=pltpu.CompilerParams(
            dimension_semantics=("arbitrary",)),
    )(emb, ws, wd, wl, nb, wa)


_C = 64


def _sc_gather(tsrc, tdst, i0, i1, pas, pad):
    e = i0.shape[0]
    n, d2 = tsrc.shape
    mesh = plsc.VectorSubcoreMesh(core_axis_name="c", subcore_axis_name="s")
    nw = 32
    chunks = e // _C

    cp = pltpu.CompilerParams()
    if "needs_layout_passes" in pltpu.CompilerParams.__dataclass_fields__:
        cp = dataclasses.replace(cp, needs_layout_passes=False)

    @functools.partial(
        pl.kernel, mesh=mesh,
        compiler_params=cp,
        out_type=(jax.ShapeDtypeStruct((e, d2), jnp.float32),
                  jax.ShapeDtypeStruct((e,), jnp.float32)),
        scratch_types=[
            pltpu.VMEM((_C,), jnp.int32), pltpu.VMEM((_C,), jnp.int32),
            pltpu.VMEM((_C, d2), jnp.float32),
            pltpu.VMEM((_C, d2), jnp.float32),
            pltpu.VMEM((_C,), jnp.float32),
            pltpu.VMEM((n,), jnp.float32), pltpu.VMEM((n,), jnp.float32),
            pltpu.SemaphoreType.DMA, pltpu.SemaphoreType.DMA,
        ],
    )
    def fn(tsrc_hbm, tdst_hbm, i0_hbm, i1_hbm, pas_hbm, pad_hbm,
           g_out, ga_out,
           idx0, idx1, bufa, bufb, gabuf, pas_v, pad_v, sema, semb):
        cid = lax.axis_index("c")
        sid = lax.axis_index("s")
        wid = sid * 2 + cid
        pltpu.sync_copy(pas_hbm, pas_v)
        pltpu.sync_copy(pad_hbm, pad_v)

        @pl.loop(wid, chunks, step=nw)
        def _chunk(c):
            base = c * _C
            pltpu.sync_copy(i0_hbm.at[pl.ds(base, _C)], idx0)
            pltpu.sync_copy(i1_hbm.at[pl.ds(base, _C)], idx1)
            cpa = pltpu.async_copy(tsrc_hbm.at[idx0], bufa, sema)
            cpb = pltpu.async_copy(tdst_hbm.at[idx1], bufb, semb)

            @pl.loop(0, _C, step=16)
            def _ga(k):
                v0 = idx0[pl.ds(k, 16)]
                v1 = idx1[pl.ds(k, 16)]
                gabuf[pl.ds(k, 16)] = (plsc.load_gather(pas_v, [v0]) +
                                       plsc.load_gather(pad_v, [v1]))

            cpa.wait()
            cpb.wait()

            @pl.loop(0, _C)
            def _row(r):
                for j in range(d2 // 16):
                    sl = pl.ds(j * 16, 16)
                    bufa[r, sl] = bufa[r, sl] + bufb[r, sl]

            pltpu.sync_copy(bufa, g_out.at[pl.ds(base, _C)])
            pltpu.sync_copy(gabuf, ga_out.at[pl.ds(base, _C)])

    return fn(tsrc, tdst, i0, i1, pas, pad)


def _edge1_body(x_ref, g_ref, ga_ref, we_ref, wae_ref, ub_ref, ab_ref,
                w2_ref, lb_ref, u_ref, s_ref, m_ref, z_ref):
    pid = pl.program_id(0)
    d = x_ref.shape[1]
    x = x_ref[...]
    g = g_ref[...]
    eu = jnp.dot(x, we_ref[...], preferred_element_type=jnp.float32)
    eu = eu + ub_ref[...]
    ea = jnp.dot(x, wae_ref[...], preferred_element_type=jnp.float32)
    pre = ea + ab_ref[...] + ga_ref[...]
    pre = jnp.where(pre >= 0, pre, 0.2 * pre)
    a = jax.nn.sigmoid(pre)
    upd = (g[:, :d] + eu) * a
    u_ref[...] = upd
    ef = jnp.dot(upd, w2_ref[...], preferred_element_type=jnp.float32)
    s = ef + lb_ref[...] + g[:, d:]
    s = jnp.where(s >= 0, s, 0.2 * s)
    s_ref[...] = s

    @pl.when(pid == 0)
    def _():
        m_ref[...] = jnp.full(m_ref.shape, -jnp.inf, jnp.float32)
        z_ref[...] = jnp.zeros(z_ref.shape, jnp.float32)

    m_old = m_ref[...]
    m_new = jnp.maximum(m_old, jnp.max(s, axis=0, keepdims=True))
    z_new = (z_ref[...] * jnp.exp(m_old - m_new) +
             jnp.sum(jnp.exp(s - m_new), axis=0, keepdims=True))
    m_ref[...] = m_new
    z_ref[...] = z_new


def _edge_pass1(x, g, ga, we, wae, ub, ab, w2, lb, be):
    e, d = x.shape
    grid = (e // be,)
    full = lambda shape: pl.BlockSpec(shape, lambda i: (0, 0))
    return pl.pallas_call(
        _edge1_body,
        grid=grid,
        in_specs=[
            pl.BlockSpec((be, d), lambda i: (i, 0)),
            pl.BlockSpec((be, 2 * d), lambda i: (i, 0)),
            pl.BlockSpec((be, 1), lambda i: (i, 0)),
            full((d, d)), full((d, 1)), full((1, d)), full((1, 1)),
            full((d, d)), full((1, d)),
        ],
        out_specs=[
            pl.BlockSpec((be, d), lambda i: (i, 0)),
            pl.BlockSpec((be, d), lambda i: (i, 0)),
            pl.BlockSpec((1, d), lambda i: (0, 0)),
            pl.BlockSpec((1, d), lambda i: (0, 0)),
        ],
        out_shape=[
            jax.ShapeDtypeStruct((e, d), jnp.float32),
            jax.ShapeDtypeStruct((e, d), jnp.float32),
            jax.ShapeDtypeStruct((1, d), jnp.float32),
            jax.ShapeDtypeStruct((1, d), jnp.float32),
        ],
        compiler_params=pltpu.CompilerParams(
            dimension_semantics=("arbitrary",)),
    )(x, g, ga, we, wae, ub, ab, w2, lb)


def _edge2_body(u_ref, s_ref, m_ref, z_ref, o_ref):
    lz = m_ref[...] + jnp.log(z_ref[...])
    o_ref[...] = u_ref[...] * jnp.exp(s_ref[...] - lz)


def _edge_pass2(u, s, m, z, be):
    e, d = u.shape
    grid = (e // be,)
    return pl.pallas_call(
        _edge2_body,
        grid=grid,
        in_specs=[
            pl.BlockSpec((be, d), lambda i: (i, 0)),
            pl.BlockSpec((be, d), lambda i: (i, 0)),
            pl.BlockSpec((1, d), lambda i: (0, 0)),
            pl.BlockSpec((1, d), lambda i: (0, 0)),
        ],
        out_specs=pl.BlockSpec((be, d), lambda i: (i, 0)),
        out_shape=jax.ShapeDtypeStruct((e, d), jnp.float32),
        compiler_params=pltpu.CompilerParams(
            dimension_semantics=("arbitrary",)),
    )(u, s, m, z)


def kernel(edge_embeddings, edge_index, edge_attr, node_embeddings,
           num_nodes, attn_W, attn_b, update_W, update_b,
           edge_lin_W, edge_lin_b, node_lin_W, node_lin_b):
    del edge_attr, num_nodes
    e, d = edge_embeddings.shape
    n = node_embeddings.shape[0]

    i0 = edge_index[0].astype(jnp.int32)
    i1 = edge_index[1].astype(jnp.int32)

    ws, wd, we = update_W[:d], update_W[d:2 * d], update_W[2 * d:]
    wa2 = jnp.concatenate([attn_W[:d], attn_W[d:2 * d]], axis=1)
    wae = attn_W[2 * d:]

    bn = 2000 if n % 2000 == 0 else n
    tsrc, tdst, p2 = _node_precompute(
        node_embeddings, ws, wd, node_lin_W,
        node_lin_b.reshape(1, d), wa2, bn)

    g, ga = _sc_gather(tsrc, tdst, i0, i1, p2[:, 0], p2[:, 1])

    be = 2000 if e % 2000 == 0 else e
    u, s, m, z = _edge_pass1(
        edge_embeddings, g, ga.reshape(e, 1), we, wae,
        update_b.reshape(1, d), attn_b.reshape(1, 1),
        edge_lin_W, edge_lin_b.reshape(1, d), be)

    return _edge_pass2(u, s, m, z, be)


# speedup vs baseline: 1.6951x; 1.1686x over previous
import dataclasses
import functools

import jax
import jax.numpy as jnp
from jax import lax
from jax.experimental import pallas as pl
from jax.experimental.pallas import tpu as pltpu
from jax.experimental.pallas import tpu_sc as plsc


def _node_body(emb_ref, ws_ref, wd_ref, wl_ref, nb_ref, wa_ref,
               tsrc_ref, tdst_ref, p2_ref):
    emb = emb_ref[...]
    d = emb.shape[1]
    pl_feat = jnp.dot(emb, wl_ref[...], preferred_element_type=jnp.float32)
    pl_feat = pl_feat + nb_ref[...]
    tsrc_ref[:, :d] = jnp.dot(emb, ws_ref[...],
                              preferred_element_type=jnp.float32)
    tsrc_ref[:, d:] = pl_feat
    tdst_ref[:, :d] = jnp.dot(emb, wd_ref[...],
                              preferred_element_type=jnp.float32)
    tdst_ref[:, d:] = pl_feat
    p2_ref[...] = jnp.dot(emb, wa_ref[...], preferred_element_type=jnp.float32)


def _node_precompute(emb, ws, wd, wl, nb, wa, bn):
    n, d = emb.shape
    grid = (n // bn,)
    full = lambda shape: pl.BlockSpec(shape, lambda i: (0, 0))
    return pl.pallas_call(
        _node_body,
        grid=grid,
        in_specs=[
            pl.BlockSpec((bn, d), lambda i: (i, 0)),
            full((d, d)), full((d, d)), full((d, d)),
            full((1, d)), full((d, 2)),
        ],
        out_specs=[
            pl.BlockSpec((bn, 2 * d), lambda i: (i, 0)),
            pl.BlockSpec((bn, 2 * d), lambda i: (i, 0)),
            pl.BlockSpec((bn, 2), lambda i: (i, 0)),
        ],
        out_shape=[
            jax.ShapeDtypeStruct((n, 2 * d), jnp.float32),
            jax.ShapeDtypeStruct((n, 2 * d), jnp.float32),
            jax.ShapeDtypeStruct((n, 2), jnp.float32),
        ],
        compiler_params=pltpu.CompilerParams(
            dimension_semantics=("arbitrary",)),
    )(emb, ws, wd, wl, nb, wa)


_C = 32


def _sc_gather(tsrc, tdst, i0, i1, pas, pad):
    e = i0.shape[0]
    n, d2 = tsrc.shape
    mesh = plsc.VectorSubcoreMesh(core_axis_name="c", subcore_axis_name="s")
    nw = 32
    chunks = e // _C

    cp = pltpu.CompilerParams()
    if "needs_layout_passes" in pltpu.CompilerParams.__dataclass_fields__:
        cp = dataclasses.replace(cp, needs_layout_passes=False)

    buf_t = pltpu.VMEM((_C, d2), jnp.float32)
    idx_t = pltpu.VMEM((_C,), jnp.int32)
    ga_t = pltpu.VMEM((_C,), jnp.float32)

    @functools.partial(
        pl.kernel, mesh=mesh,
        compiler_params=cp,
        out_type=(jax.ShapeDtypeStruct((e, d2), jnp.float32),
                  jax.ShapeDtypeStruct((e,), jnp.float32)),
        scratch_types=[
            idx_t, idx_t, idx_t, idx_t,
            buf_t, buf_t, buf_t, buf_t,
            ga_t, ga_t,
            pltpu.VMEM((n,), jnp.float32), pltpu.VMEM((n,), jnp.float32),
            pltpu.SemaphoreType.DMA, pltpu.SemaphoreType.DMA,
            pltpu.SemaphoreType.DMA, pltpu.SemaphoreType.DMA,
        ],
    )
    def fn(tsrc_hbm, tdst_hbm, i0_hbm, i1_hbm, pas_hbm, pad_hbm,
           g_out, ga_out,
           idx0a, idx1a, idx0b, idx1b, bsa, bda, bsb, bdb, gaa, gab,
           pas_v, pad_v, sema, semb, semwa, semwb):
        cid = lax.axis_index("c")
        sid = lax.axis_index("s")
        wid = sid * 2 + cid
        pltpu.sync_copy(pas_hbm, pas_v)
        pltpu.sync_copy(pad_hbm, pad_v)

        def drain(bufs, bufd, gabuf, semw):
            pltpu.make_async_copy(bufs, g_out.at[pl.ds(0, _C)], semw).wait()
            pltpu.make_async_copy(gabuf, ga_out.at[pl.ds(0, _C)], semw).wait()
            del bufd

        def ga_compute(idx0, idx1, gabuf):
            @pl.loop(0, _C, step=16)
            def _ga(k):
                v0 = idx0[pl.ds(k, 16)]
                v1 = idx1[pl.ds(k, 16)]
                gabuf[pl.ds(k, 16)] = (plsc.load_gather(pas_v, [v0]) +
                                       plsc.load_gather(pad_v, [v1]))

        def add_rows(bufs, bufd):
            @pl.loop(0, _C)
            def _row(r):
                for j in range(d2 // 16):
                    sl = pl.ds(j * 16, 16)
                    bufs[r, sl] = bufs[r, sl] + bufd[r, sl]

        def writeback(c, bufs, gabuf, semw):
            base = c * _C
            pltpu.async_copy(bufs, g_out.at[pl.ds(base, _C)], semw)
            pltpu.async_copy(gabuf, ga_out.at[pl.ds(base, _C)], semw)

        @pl.loop(wid, chunks, step=2 * nw)
        def _pair(c0):
            c1 = c0 + nw

            @pl.when(c0 > wid)
            def _():
                drain(bsa, bda, gaa, semwa)
            base0 = c0 * _C
            pltpu.sync_copy(i0_hbm.at[pl.ds(base0, _C)], idx0a)
            pltpu.sync_copy(i1_hbm.at[pl.ds(base0, _C)], idx1a)
            cpa0 = pltpu.async_copy(tsrc_hbm.at[idx0a], bsa, sema)
            cpa1 = pltpu.async_copy(tdst_hbm.at[idx1a], bda, sema)

            @pl.when(c1 < chunks)
            def _():
                @pl.when(c0 > wid)
                def _():
                    drain(bsb, bdb, gab, semwb)
                base1 = c1 * _C
                pltpu.sync_copy(i0_hbm.at[pl.ds(base1, _C)], idx0b)
                pltpu.sync_copy(i1_hbm.at[pl.ds(base1, _C)], idx1b)
                pltpu.async_copy(tsrc_hbm.at[idx0b], bsb, semb)
                pltpu.async_copy(tdst_hbm.at[idx1b], bdb, semb)

            ga_compute(idx0a, idx1a, gaa)
            cpa0.wait()
            cpa1.wait()
            add_rows(bsa, bda)
            writeback(c0, bsa, gaa, semwa)

            @pl.when(c1 < chunks)
            def _():
                ga_compute(idx0b, idx1b, gab)
                pltpu.make_async_copy(tsrc_hbm.at[idx0b], bsb, semb).wait()
                pltpu.make_async_copy(tdst_hbm.at[idx1b], bdb, semb).wait()
                add_rows(bsb, bdb)
                writeback(c1, bsb, gab, semwb)

        kk = (chunks - wid + nw - 1) // nw
        drain(bsa, bda, gaa, semwa)

        @pl.when((kk % 2 == 0) & (kk > 0))
        def _():
            drain(bsb, bdb, gab, semwb)

    return fn(tsrc, tdst, i0, i1, pas, pad)


def _edge1_body(x_ref, g_ref, ga_ref, we_ref, wae_ref, ub_ref, ab_ref,
                w2_ref, lb_ref, u_ref, s_ref, m_ref, z_ref):
    pid = pl.program_id(0)
    d = x_ref.shape[1]
    x = x_ref[...]
    g = g_ref[...]
    eu = jnp.dot(x, we_ref[...], preferred_element_type=jnp.float32)
    eu = eu + ub_ref[...]
    ea = jnp.dot(x, wae_ref[...], preferred_element_type=jnp.float32)
    pre = ea + ab_ref[...] + ga_ref[...]
    pre = jnp.where(pre >= 0, pre, 0.2 * pre)
    a = jax.nn.sigmoid(pre)
    upd = (g[:, :d] + eu) * a
    u_ref[...] = upd
    ef = jnp.dot(upd, w2_ref[...], preferred_element_type=jnp.float32)
    s = ef + lb_ref[...] + g[:, d:]
    s = jnp.where(s >= 0, s, 0.2 * s)
    s_ref[...] = s

    @pl.when(pid == 0)
    def _():
        m_ref[...] = jnp.full(m_ref.shape, -jnp.inf, jnp.float32)
        z_ref[...] = jnp.zeros(z_ref.shape, jnp.float32)

    m_old = m_ref[...]
    m_new = jnp.maximum(m_old, jnp.max(s, axis=0, keepdims=True))
    z_new = (z_ref[...] * jnp.exp(m_old - m_new) +
             jnp.sum(jnp.exp(s - m_new), axis=0, keepdims=True))
    m_ref[...] = m_new
    z_ref[...] = z_new


def _edge_pass1(x, g, ga, we, wae, ub, ab, w2, lb, be):
    e, d = x.shape
    grid = (e // be,)
    full = lambda shape: pl.BlockSpec(shape, lambda i: (0, 0))
    return pl.pallas_call(
        _edge1_body,
        grid=grid,
        in_specs=[
            pl.BlockSpec((be, d), lambda i: (i, 0)),
            pl.BlockSpec((be, 2 * d), lambda i: (i, 0)),
            pl.BlockSpec((be, 1), lambda i: (i, 0)),
            full((d, d)), full((d, 1)), full((1, d)), full((1, 1)),
            full((d, d)), full((1, d)),
        ],
        out_specs=[
            pl.BlockSpec((be, d), lambda i: (i, 0)),
            pl.BlockSpec((be, d), lambda i: (i, 0)),
            pl.BlockSpec((1, d), lambda i: (0, 0)),
            pl.BlockSpec((1, d), lambda i: (0, 0)),
        ],
        out_shape=[
            jax.ShapeDtypeStruct((e, d), jnp.float32),
            jax.ShapeDtypeStruct((e, d), jnp.float32),
            jax.ShapeDtypeStruct((1, d), jnp.float32),
            jax.ShapeDtypeStruct((1, d), jnp.float32),
        ],
        compiler_params=pltpu.CompilerParams(
            dimension_semantics=("arbitrary",)),
    )(x, g, ga, we, wae, ub, ab, w2, lb)


def _edge2_body(u_ref, s_ref, m_ref, z_ref, o_ref):
    lz = m_ref[...] + jnp.log(z_ref[...])
    o_ref[...] = u_ref[...] * jnp.exp(s_ref[...] - lz)


def _edge_pass2(u, s, m, z, be):
    e, d = u.shape
    grid = (e // be,)
    return pl.pallas_call(
        _edge2_body,
        grid=grid,
        in_specs=[
            pl.BlockSpec((be, d), lambda i: (i, 0)),
            pl.BlockSpec((be, d), lambda i: (i, 0)),
            pl.BlockSpec((1, d), lambda i: (0, 0)),
            pl.BlockSpec((1, d), lambda i: (0, 0)),
        ],
        out_specs=pl.BlockSpec((be, d), lambda i: (i, 0)),
        out_shape=jax.ShapeDtypeStruct((e, d), jnp.float32),
        compiler_params=pltpu.CompilerParams(
            dimension_semantics=("arbitrary",)),
    )(u, s, m, z)


def kernel(edge_embeddings, edge_index, edge_attr, node_embeddings,
           num_nodes, attn_W, attn_b, update_W, update_b,
           edge_lin_W, edge_lin_b, node_lin_W, node_lin_b):
    del edge_attr, num_nodes
    e, d = edge_embeddings.shape
    n = node_embeddings.shape[0]

    i0 = edge_index[0].astype(jnp.int32)
    i1 = edge_index[1].astype(jnp.int32)

    ws, wd, we = update_W[:d], update_W[d:2 * d], update_W[2 * d:]
    wa2 = jnp.concatenate([attn_W[:d], attn_W[d:2 * d]], axis=1)
    wae = attn_W[2 * d:]

    bn = 2000 if n % 2000 == 0 else n
    tsrc, tdst, p2 = _node_precompute(
        node_embeddings, ws, wd, node_lin_W,
        node_lin_b.reshape(1, d), wa2, bn)

    g, ga = _sc_gather(tsrc, tdst, i0, i1, p2[:, 0], p2[:, 1])

    be = 2000 if e % 2000 == 0 else e
    u, s, m, z = _edge_pass1(
        edge_embeddings, g, ga.reshape(e, 1), we, wae,
        update_b.reshape(1, d), attn_b.reshape(1, 1),
        edge_lin_W, edge_lin_b.reshape(1, d), be)

    return _edge_pass2(u, s, m, z, be)


# speedup vs baseline: 1.7433x; 1.0284x over previous
import dataclasses
import functools

import jax
import jax.numpy as jnp
from jax import lax
from jax.experimental import pallas as pl
from jax.experimental.pallas import tpu as pltpu
from jax.experimental.pallas import tpu_sc as plsc


def _node_body(emb_ref, ws_ref, wd_ref, wl_ref, nb_ref, wa_ref,
               tsrc_ref, tdst_ref, p2_ref):
    emb = emb_ref[...]
    d = emb.shape[1]
    pl_feat = jnp.dot(emb, wl_ref[...], preferred_element_type=jnp.float32)
    pl_feat = pl_feat + nb_ref[...]
    tsrc_ref[:, :d] = jnp.dot(emb, ws_ref[...],
                              preferred_element_type=jnp.float32)
    tsrc_ref[:, d:] = pl_feat
    tdst_ref[:, :d] = jnp.dot(emb, wd_ref[...],
                              preferred_element_type=jnp.float32)
    tdst_ref[:, d:] = pl_feat
    p2_ref[...] = jnp.dot(emb, wa_ref[...], preferred_element_type=jnp.float32)


def _node_precompute(emb, ws, wd, wl, nb, wa, bn):
    n, d = emb.shape
    grid = (n // bn,)
    full = lambda shape: pl.BlockSpec(shape, lambda i: (0, 0))
    return pl.pallas_call(
        _node_body,
        grid=grid,
        in_specs=[
            pl.BlockSpec((bn, d), lambda i: (i, 0)),
            full((d, d)), full((d, d)), full((d, d)),
            full((1, d)), full((d, 2)),
        ],
        out_specs=[
            pl.BlockSpec((bn, 2 * d), lambda i: (i, 0)),
            pl.BlockSpec((bn, 2 * d), lambda i: (i, 0)),
            pl.BlockSpec((bn, 2), lambda i: (i, 0)),
        ],
        out_shape=[
            jax.ShapeDtypeStruct((n, 2 * d), jnp.float32),
            jax.ShapeDtypeStruct((n, 2 * d), jnp.float32),
            jax.ShapeDtypeStruct((n, 2), jnp.float32),
        ],
        compiler_params=pltpu.CompilerParams(
            dimension_semantics=("arbitrary",)),
    )(emb, ws, wd, wl, nb, wa)


_C = 32


def _sc_gather(tsrc, tdst, i0, i1, pas, pad):
    e = i0.shape[0]
    n, d2 = tsrc.shape
    mesh = plsc.VectorSubcoreMesh(core_axis_name="c", subcore_axis_name="s")
    nw = 32
    chunks = e // _C

    cp = pltpu.CompilerParams()
    if "needs_layout_passes" in pltpu.CompilerParams.__dataclass_fields__:
        cp = dataclasses.replace(cp, needs_layout_passes=False)

    per = -(-(e // nw) // _C) * _C
    tail = e - (nw - 1) * per
    assert tail > 0 and tail % _C == 0 and per % 8 == 0

    buf_t = pltpu.VMEM((_C, d2), jnp.float32)
    ga_t = pltpu.VMEM((_C,), jnp.float32)

    @functools.partial(
        pl.kernel, mesh=mesh,
        compiler_params=cp,
        out_type=(jax.ShapeDtypeStruct((e, d2), jnp.float32),
                  jax.ShapeDtypeStruct((e,), jnp.float32)),
        scratch_types=[
            pltpu.VMEM((per,), jnp.int32), pltpu.VMEM((per,), jnp.int32),
            buf_t, buf_t, buf_t, buf_t,
            ga_t, ga_t,
            pltpu.VMEM((n,), jnp.float32), pltpu.VMEM((n,), jnp.float32),
            pltpu.SemaphoreType.DMA, pltpu.SemaphoreType.DMA,
            pltpu.SemaphoreType.DMA, pltpu.SemaphoreType.DMA,
        ],
    )
    def fn(tsrc_hbm, tdst_hbm, i0_hbm, i1_hbm, pas_hbm, pad_hbm,
           g_out, ga_out,
           idx0_v, idx1_v, bsa, bda, bsb, bdb, gaa, gab,
           pas_v, pad_v, sema, semb, semwa, semwb):
        cid = lax.axis_index("c")
        sid = lax.axis_index("s")
        wid = sid * 2 + cid
        base_w = wid * per
        nch = jnp.where(wid == nw - 1, tail // _C, per // _C)
        pltpu.sync_copy(pas_hbm, pas_v)
        pltpu.sync_copy(pad_hbm, pad_v)

        @pl.when(wid < nw - 1)
        def _():
            pltpu.sync_copy(i0_hbm.at[pl.ds(base_w, per)], idx0_v)
            pltpu.sync_copy(i1_hbm.at[pl.ds(base_w, per)], idx1_v)

        @pl.when(wid == nw - 1)
        def _():
            pltpu.sync_copy(i0_hbm.at[pl.ds(base_w, tail)],
                            idx0_v.at[pl.ds(0, tail)])
            pltpu.sync_copy(i1_hbm.at[pl.ds(base_w, tail)],
                            idx1_v.at[pl.ds(0, tail)])

        def drain(bufs, gabuf, semw):
            pltpu.make_async_copy(bufs, g_out.at[pl.ds(0, _C)], semw).wait()
            pltpu.make_async_copy(gabuf, ga_out.at[pl.ds(0, _C)], semw).wait()

        def issue(t, bufs, bufd, sem):
            off = t * _C
            return (
                pltpu.async_copy(tsrc_hbm.at[idx0_v.at[pl.ds(off, _C)]],
                                 bufs, sem),
                pltpu.async_copy(tdst_hbm.at[idx1_v.at[pl.ds(off, _C)]],
                                 bufd, sem),
            )

        def wait_issue(bufs, bufd, sem):
            pltpu.make_async_copy(tsrc_hbm.at[idx0_v.at[pl.ds(0, _C)]],
                                  bufs, sem).wait()
            pltpu.make_async_copy(tdst_hbm.at[idx1_v.at[pl.ds(0, _C)]],
                                  bufd, sem).wait()

        def ga_compute(t, gabuf):
            off = t * _C

            @pl.loop(0, _C, step=16)
            def _ga(k):
                v0 = idx0_v[pl.ds(off + k, 16)]
                v1 = idx1_v[pl.ds(off + k, 16)]
                gabuf[pl.ds(k, 16)] = (plsc.load_gather(pas_v, [v0]) +
                                       plsc.load_gather(pad_v, [v1]))

        def add_rows(bufs, bufd):
            @pl.loop(0, _C)
            def _row(r):
                for j in range(d2 // 16):
                    sl = pl.ds(j * 16, 16)
                    bufs[r, sl] = bufs[r, sl] + bufd[r, sl]

        def writeback(t, bufs, gabuf, semw):
            base = base_w + t * _C
            pltpu.async_copy(bufs, g_out.at[pl.ds(base, _C)], semw)
            pltpu.async_copy(gabuf, ga_out.at[pl.ds(base, _C)], semw)

        @pl.loop(0, per // _C, step=2)
        def _pair(t0):
            t1 = t0 + 1

            @pl.when(t0 < nch)
            def _():
                @pl.when(t0 > 0)
                def _():
                    drain(bsa, gaa, semwa)
                issue(t0, bsa, bda, sema)

                @pl.when(t1 < nch)
                def _():
                    @pl.when(t0 > 0)
                    def _():
                        drain(bsb, gab, semwb)
                    issue(t1, bsb, bdb, semb)

                ga_compute(t0, gaa)
                wait_issue(bsa, bda, sema)
                add_rows(bsa, bda)
                writeback(t0, bsa, gaa, semwa)

                @pl.when(t1 < nch)
                def _():
                    ga_compute(t1, gab)
                    wait_issue(bsb, bdb, semb)
                    add_rows(bsb, bdb)
                    writeback(t1, bsb, gab, semwb)

        drain(bsa, gaa, semwa)

        @pl.when(nch >= 2)
        def _():
            drain(bsb, gab, semwb)

    return fn(tsrc, tdst, i0, i1, pas, pad)


def _edge1_body(x_ref, g_ref, ga_ref, we_ref, wae_ref, ub_ref, ab_ref,
                w2_ref, lb_ref, u_ref, s_ref, m_ref, z_ref):
    pid = pl.program_id(0)
    d = x_ref.shape[1]
    x = x_ref[...]
    g = g_ref[...]
    eu = jnp.dot(x, we_ref[...], preferred_element_type=jnp.float32)
    eu = eu + ub_ref[...]
    ea = jnp.dot(x, wae_ref[...], preferred_element_type=jnp.float32)
    pre = ea + ab_ref[...] + ga_ref[...]
    pre = jnp.where(pre >= 0, pre, 0.2 * pre)
    a = jax.nn.sigmoid(pre)
    upd = (g[:, :d] + eu) * a
    u_ref[...] = upd
    ef = jnp.dot(upd, w2_ref[...], preferred_element_type=jnp.float32)
    s = ef + lb_ref[...] + g[:, d:]
    s = jnp.where(s >= 0, s, 0.2 * s)
    s_ref[...] = s

    @pl.when(pid == 0)
    def _():
        m_ref[...] = jnp.full(m_ref.shape, -jnp.inf, jnp.float32)
        z_ref[...] = jnp.zeros(z_ref.shape, jnp.float32)

    m_old = m_ref[...]
    m_new = jnp.maximum(m_old, jnp.max(s, axis=0, keepdims=True))
    z_new = (z_ref[...] * jnp.exp(m_old - m_new) +
             jnp.sum(jnp.exp(s - m_new), axis=0, keepdims=True))
    m_ref[...] = m_new
    z_ref[...] = z_new


def _edge_pass1(x, g, ga, we, wae, ub, ab, w2, lb, be):
    e, d = x.shape
    grid = (e // be,)
    full = lambda shape: pl.BlockSpec(shape, lambda i: (0, 0))
    return pl.pallas_call(
        _edge1_body,
        grid=grid,
        in_specs=[
            pl.BlockSpec((be, d), lambda i: (i, 0)),
            pl.BlockSpec((be, 2 * d), lambda i: (i, 0)),
            pl.BlockSpec((be, 1), lambda i: (i, 0)),
            full((d, d)), full((d, 1)), full((1, d)), full((1, 1)),
            full((d, d)), full((1, d)),
        ],
        out_specs=[
            pl.BlockSpec((be, d), lambda i: (i, 0)),
            pl.BlockSpec((be, d), lambda i: (i, 0)),
            pl.BlockSpec((1, d), lambda i: (0, 0)),
            pl.BlockSpec((1, d), lambda i: (0, 0)),
        ],
        out_shape=[
            jax.ShapeDtypeStruct((e, d), jnp.float32),
            jax.ShapeDtypeStruct((e, d), jnp.float32),
            jax.ShapeDtypeStruct((1, d), jnp.float32),
            jax.ShapeDtypeStruct((1, d), jnp.float32),
        ],
        compiler_params=pltpu.CompilerParams(
            dimension_semantics=("arbitrary",)),
    )(x, g, ga, we, wae, ub, ab, w2, lb)


def _edge2_body(u_ref, s_ref, m_ref, z_ref, o_ref):
    lz = m_ref[...] + jnp.log(z_ref[...])
    o_ref[...] = u_ref[...] * jnp.exp(s_ref[...] - lz)


def _edge_pass2(u, s, m, z, be):
    e, d = u.shape
    grid = (e // be,)
    return pl.pallas_call(
        _edge2_body,
        grid=grid,
        in_specs=[
            pl.BlockSpec((be, d), lambda i: (i, 0)),
            pl.BlockSpec((be, d), lambda i: (i, 0)),
            pl.BlockSpec((1, d), lambda i: (0, 0)),
            pl.BlockSpec((1, d), lambda i: (0, 0)),
        ],
        out_specs=pl.BlockSpec((be, d), lambda i: (i, 0)),
        out_shape=jax.ShapeDtypeStruct((e, d), jnp.float32),
        compiler_params=pltpu.CompilerParams(
            dimension_semantics=("arbitrary",)),
    )(u, s, m, z)


def kernel(edge_embeddings, edge_index, edge_attr, node_embeddings,
           num_nodes, attn_W, attn_b, update_W, update_b,
           edge_lin_W, edge_lin_b, node_lin_W, node_lin_b):
    del edge_attr, num_nodes
    e, d = edge_embeddings.shape
    n = node_embeddings.shape[0]

    i0 = edge_index[0].astype(jnp.int32)
    i1 = edge_index[1].astype(jnp.int32)

    ws, wd, we = update_W[:d], update_W[d:2 * d], update_W[2 * d:]
    wa2 = jnp.concatenate([attn_W[:d], attn_W[d:2 * d]], axis=1)
    wae = attn_W[2 * d:]

    bn = 2000 if n % 2000 == 0 else n
    tsrc, tdst, p2 = _node_precompute(
        node_embeddings, ws, wd, node_lin_W,
        node_lin_b.reshape(1, d), wa2, bn)

    g, ga = _sc_gather(tsrc, tdst, i0, i1, p2[:, 0], p2[:, 1])

    be = 2000 if e % 2000 == 0 else e
    u, s, m, z = _edge_pass1(
        edge_embeddings, g, ga.reshape(e, 1), we, wae,
        update_b.reshape(1, d), attn_b.reshape(1, 1),
        edge_lin_W, edge_lin_b.reshape(1, d), be)

    return _edge_pass2(u, s, m, z, be)


# speedup vs baseline: 1.9010x; 1.0905x over previous
import dataclasses
import functools

import jax
import jax.numpy as jnp
from jax import lax
from jax.experimental import pallas as pl
from jax.experimental.pallas import tpu as pltpu
from jax.experimental.pallas import tpu_sc as plsc


def _node_body(emb_ref, ws_ref, wd_ref, wl_ref, nb_ref, wa_ref,
               tsrc_ref, tdst_ref, p2_ref):
    emb = emb_ref[...]
    d = emb.shape[1]
    pl_feat = jnp.dot(emb, wl_ref[...], preferred_element_type=jnp.float32)
    pl_feat = pl_feat + nb_ref[...]
    tsrc_ref[:, :d] = jnp.dot(emb, ws_ref[...],
                              preferred_element_type=jnp.float32)
    tsrc_ref[:, d:] = pl_feat
    tdst_ref[:, :d] = jnp.dot(emb, wd_ref[...],
                              preferred_element_type=jnp.float32)
    tdst_ref[:, d:] = pl_feat
    p2_ref[...] = jnp.dot(emb, wa_ref[...], preferred_element_type=jnp.float32)


def _node_precompute(emb, ws, wd, wl, nb, wa, bn):
    n, d = emb.shape
    grid = (n // bn,)
    full = lambda shape: pl.BlockSpec(shape, lambda i: (0, 0))
    return pl.pallas_call(
        _node_body,
        grid=grid,
        in_specs=[
            pl.BlockSpec((bn, d), lambda i: (i, 0)),
            full((d, d)), full((d, d)), full((d, d)),
            full((1, d)), full((d, 2)),
        ],
        out_specs=[
            pl.BlockSpec((bn, 2 * d), lambda i: (i, 0)),
            pl.BlockSpec((bn, 2 * d), lambda i: (i, 0)),
            pl.BlockSpec((bn, 2), lambda i: (i, 0)),
        ],
        out_shape=[
            jax.ShapeDtypeStruct((n, 2 * d), jnp.float32),
            jax.ShapeDtypeStruct((n, 2 * d), jnp.float32),
            jax.ShapeDtypeStruct((n, 2), jnp.float32),
        ],
        compiler_params=pltpu.CompilerParams(
            dimension_semantics=("arbitrary",)),
    )(emb, ws, wd, wl, nb, wa)


_C = 32


def _sc_gather(tsrc, tdst, i0, i1, pas, pad):
    e = i0.shape[0]
    n, d2 = tsrc.shape
    mesh = plsc.VectorSubcoreMesh(core_axis_name="c", subcore_axis_name="s")
    nw = 32
    chunks = e // _C

    cp = pltpu.CompilerParams()
    if "needs_layout_passes" in pltpu.CompilerParams.__dataclass_fields__:
        cp = dataclasses.replace(cp, needs_layout_passes=False)

    per = -(-(e // nw) // _C) * _C
    tail = e - (nw - 1) * per
    assert tail > 0 and tail % _C == 0 and per % 8 == 0

    buf_t = pltpu.VMEM((_C, d2), jnp.float32)
    ga_t = pltpu.VMEM((_C,), jnp.float32)

    @functools.partial(
        pl.kernel, mesh=mesh,
        compiler_params=cp,
        out_type=(jax.ShapeDtypeStruct((e, d2), jnp.float32),
                  jax.ShapeDtypeStruct((e,), jnp.float32)),
        scratch_types=[
            pltpu.VMEM((per,), jnp.int32), pltpu.VMEM((per,), jnp.int32),
            buf_t, buf_t, buf_t, buf_t,
            ga_t, ga_t,
            pltpu.VMEM((n,), jnp.float32), pltpu.VMEM((n,), jnp.float32),
            pltpu.SemaphoreType.DMA, pltpu.SemaphoreType.DMA,
            pltpu.SemaphoreType.DMA, pltpu.SemaphoreType.DMA,
        ],
    )
    def fn(tsrc_hbm, tdst_hbm, i0_hbm, i1_hbm, pas_hbm, pad_hbm,
           g_out, ga_out,
           idx0_v, idx1_v, bsa, bda, bsb, bdb, gaa, gab,
           pas_v, pad_v, sema, semb, semwa, semwb):
        cid = lax.axis_index("c")
        sid = lax.axis_index("s")
        wid = sid * 2 + cid
        base_w = wid * per
        nch = jnp.where(wid == nw - 1, tail // _C, per // _C)
        pltpu.sync_copy(pas_hbm, pas_v)
        pltpu.sync_copy(pad_hbm, pad_v)

        @pl.when(wid < nw - 1)
        def _():
            pltpu.sync_copy(i0_hbm.at[pl.ds(base_w, per)], idx0_v)
            pltpu.sync_copy(i1_hbm.at[pl.ds(base_w, per)], idx1_v)

        @pl.when(wid == nw - 1)
        def _():
            pltpu.sync_copy(i0_hbm.at[pl.ds(base_w, tail)],
                            idx0_v.at[pl.ds(0, tail)])
            pltpu.sync_copy(i1_hbm.at[pl.ds(base_w, tail)],
                            idx1_v.at[pl.ds(0, tail)])

        def drain(bufs, gabuf, semw):
            pltpu.make_async_copy(bufs, g_out.at[pl.ds(0, _C)], semw).wait()
            pltpu.make_async_copy(gabuf, ga_out.at[pl.ds(0, _C)], semw).wait()

        def issue(t, bufs, bufd, sem):
            off = t * _C
            return (
                pltpu.async_copy(tsrc_hbm.at[idx0_v.at[pl.ds(off, _C)]],
                                 bufs, sem),
                pltpu.async_copy(tdst_hbm.at[idx1_v.at[pl.ds(off, _C)]],
                                 bufd, sem),
            )

        def wait_issue(bufs, bufd, sem):
            pltpu.make_async_copy(tsrc_hbm.at[idx0_v.at[pl.ds(0, _C)]],
                                  bufs, sem).wait()
            pltpu.make_async_copy(tdst_hbm.at[idx1_v.at[pl.ds(0, _C)]],
                                  bufd, sem).wait()

        def ga_compute(t, gabuf):
            off = t * _C

            @pl.loop(0, _C, step=16)
            def _ga(k):
                v0 = idx0_v[pl.ds(off + k, 16)]
                v1 = idx1_v[pl.ds(off + k, 16)]
                gabuf[pl.ds(k, 16)] = (plsc.load_gather(pas_v, [v0]) +
                                       plsc.load_gather(pad_v, [v1]))

        def add_rows(bufs, bufd):
            @pl.loop(0, _C)
            def _row(r):
                for j in range(d2 // 16):
                    sl = pl.ds(j * 16, 16)
                    bufs[r, sl] = bufs[r, sl] + bufd[r, sl]

        def writeback(t, bufs, gabuf, semw):
            base = base_w + t * _C
            pltpu.async_copy(bufs, g_out.at[pl.ds(base, _C)], semw)
            pltpu.async_copy(gabuf, ga_out.at[pl.ds(base, _C)], semw)

        @pl.loop(0, per // _C, step=2)
        def _pair(t0):
            t1 = t0 + 1

            @pl.when(t0 < nch)
            def _():
                @pl.when(t0 > 0)
                def _():
                    drain(bsa, gaa, semwa)
                issue(t0, bsa, bda, sema)

                @pl.when(t1 < nch)
                def _():
                    @pl.when(t0 > 0)
                    def _():
                        drain(bsb, gab, semwb)
                    issue(t1, bsb, bdb, semb)

                ga_compute(t0, gaa)
                wait_issue(bsa, bda, sema)
                add_rows(bsa, bda)
                writeback(t0, bsa, gaa, semwa)

                @pl.when(t1 < nch)
                def _():
                    ga_compute(t1, gab)
                    wait_issue(bsb, bdb, semb)
                    add_rows(bsb, bdb)
                    writeback(t1, bsb, gab, semwb)

        drain(bsa, gaa, semwa)

        @pl.when(nch >= 2)
        def _():
            drain(bsb, gab, semwb)

    return fn(tsrc, tdst, i0, i1, pas, pad)


def _edge1_body(x_ref, g_ref, ga_ref, we_ref, wae_ref, ub_ref, ab_ref,
                w2_ref, lb_ref, xo_ref, mrow_ref, m_ref, z_ref):
    pid = pl.program_id(0)
    d = x_ref.shape[1]
    x = x_ref[...]
    g = g_ref[...]
    eu = jnp.dot(x, we_ref[...], preferred_element_type=jnp.float32)
    eu = eu + ub_ref[...]
    ea = jnp.dot(x, wae_ref[...], preferred_element_type=jnp.float32)
    pre = ea + ab_ref[...] + ga_ref[...]
    pre = jnp.where(pre >= 0, pre, 0.2 * pre)
    a = jax.nn.sigmoid(pre)
    upd = (g[:, :d] + eu) * a
    ef = jnp.dot(upd, w2_ref[...], preferred_element_type=jnp.float32)
    s = ef + lb_ref[...] + g[:, d:]
    s = jnp.where(s >= 0, s, 0.2 * s)

    @pl.when(pid == 0)
    def _():
        m_ref[...] = jnp.full(m_ref.shape, -jnp.inf, jnp.float32)
        z_ref[...] = jnp.zeros(z_ref.shape, jnp.float32)

    m_old = m_ref[...]
    m_new = jnp.maximum(m_old, jnp.max(s, axis=0, keepdims=True))
    expv = jnp.exp(s - m_new)
    z_new = z_ref[...] * jnp.exp(m_old - m_new) + jnp.sum(
        expv, axis=0, keepdims=True)
    m_ref[...] = m_new
    z_ref[...] = z_new
    xo_ref[...] = upd * expv
    mrow_ref[...] = m_new.reshape(mrow_ref.shape)


def _edge_pass1(x, g, ga, we, wae, ub, ab, w2, lb, be):
    e, d = x.shape
    grid = (e // be,)
    full = lambda shape: pl.BlockSpec(shape, lambda i: (0, 0))
    return pl.pallas_call(
        _edge1_body,
        grid=grid,
        in_specs=[
            pl.BlockSpec((be, d), lambda i: (i, 0)),
            pl.BlockSpec((be, 2 * d), lambda i: (i, 0)),
            pl.BlockSpec((be, 1), lambda i: (i, 0)),
            full((d, d)), full((d, 1)), full((1, d)), full((1, 1)),
            full((d, d)), full((1, d)),
        ],
        out_specs=[
            pl.BlockSpec((be, d), lambda i: (i, 0)),
            pl.BlockSpec((1, 1, d), lambda i: (i, 0, 0)),
            pl.BlockSpec((1, d), lambda i: (0, 0)),
            pl.BlockSpec((1, d), lambda i: (0, 0)),
        ],
        out_shape=[
            jax.ShapeDtypeStruct((e, d), jnp.float32),
            jax.ShapeDtypeStruct((e // be, 1, d), jnp.float32),
            jax.ShapeDtypeStruct((1, d), jnp.float32),
            jax.ShapeDtypeStruct((1, d), jnp.float32),
        ],
        compiler_params=pltpu.CompilerParams(
            dimension_semantics=("arbitrary",)),
    )(x, g, ga, we, wae, ub, ab, w2, lb)


def _edge2_body(x_ref, mrow_ref, m_ref, z_ref, o_ref):
    scale = jnp.exp(mrow_ref[0] - m_ref[...]) / z_ref[...]
    o_ref[...] = x_ref[...] * scale


def _edge_pass2(xs, mrow, m, z, be):
    e, d = xs.shape
    grid = (e // be,)
    return pl.pallas_call(
        _edge2_body,
        grid=grid,
        in_specs=[
            pl.BlockSpec((be, d), lambda i: (i, 0)),
            pl.BlockSpec((1, 1, d), lambda i: (i, 0, 0)),
            pl.BlockSpec((1, d), lambda i: (0, 0)),
            pl.BlockSpec((1, d), lambda i: (0, 0)),
        ],
        out_specs=pl.BlockSpec((be, d), lambda i: (i, 0)),
        out_shape=jax.ShapeDtypeStruct((e, d), jnp.float32),
        compiler_params=pltpu.CompilerParams(
            dimension_semantics=("arbitrary",)),
    )(xs, mrow, m, z)


def kernel(edge_embeddings, edge_index, edge_attr, node_embeddings,
           num_nodes, attn_W, attn_b, update_W, update_b,
           edge_lin_W, edge_lin_b, node_lin_W, node_lin_b):
    del edge_attr, num_nodes
    e, d = edge_embeddings.shape
    n = node_embeddings.shape[0]

    i0 = edge_index[0].astype(jnp.int32)
    i1 = edge_index[1].astype(jnp.int32)

    ws, wd, we = update_W[:d], update_W[d:2 * d], update_W[2 * d:]
    wa2 = jnp.concatenate([attn_W[:d], attn_W[d:2 * d]], axis=1)
    wae = attn_W[2 * d:]

    bn = 2000 if n % 2000 == 0 else n
    tsrc, tdst, p2 = _node_precompute(
        node_embeddings, ws, wd, node_lin_W,
        node_lin_b.reshape(1, d), wa2, bn)

    g, ga = _sc_gather(tsrc, tdst, i0, i1, p2[:, 0], p2[:, 1])

    be = 2000 if e % 2000 == 0 else e
    xs, mrow, m, z = _edge_pass1(
        edge_embeddings, g, ga.reshape(e, 1), we, wae,
        update_b.reshape(1, d), attn_b.reshape(1, 1),
        edge_lin_W, edge_lin_b.reshape(1, d), be)

    return _edge_pass2(xs, mrow, m, z, be)


# speedup vs baseline: 1.9075x; 1.0034x over previous
import dataclasses
import functools

import jax
import jax.numpy as jnp
from jax import lax
from jax.experimental import pallas as pl
from jax.experimental.pallas import tpu as pltpu
from jax.experimental.pallas import tpu_sc as plsc


def _node_body(emb_ref, ws_ref, wd_ref, wl_ref, nb_ref, wa_ref,
               tsrc_ref, tdst_ref, p2_ref):
    emb = emb_ref[...]
    d = emb.shape[1]
    pl_feat = jnp.dot(emb, wl_ref[...], preferred_element_type=jnp.float32)
    pl_feat = pl_feat + nb_ref[...]
    tsrc_ref[:, :d] = jnp.dot(emb, ws_ref[...],
                              preferred_element_type=jnp.float32)
    tsrc_ref[:, d:] = pl_feat
    tdst_ref[:, :d] = jnp.dot(emb, wd_ref[...],
                              preferred_element_type=jnp.float32)
    tdst_ref[:, d:] = pl_feat
    p2_ref[...] = jnp.dot(emb, wa_ref[...], preferred_element_type=jnp.float32)


def _node_precompute(emb, ws, wd, wl, nb, wa, bn):
    n, d = emb.shape
    grid = (n // bn,)
    full = lambda shape: pl.BlockSpec(shape, lambda i: (0, 0))
    return pl.pallas_call(
        _node_body,
        grid=grid,
        in_specs=[
            pl.BlockSpec((bn, d), lambda i: (i, 0)),
            full((d, d)), full((d, d)), full((d, d)),
            full((1, d)), full((d, 2)),
        ],
        out_specs=[
            pl.BlockSpec((bn, 2 * d), lambda i: (i, 0)),
            pl.BlockSpec((bn, 2 * d), lambda i: (i, 0)),
            pl.BlockSpec((bn, 2), lambda i: (i, 0)),
        ],
        out_shape=[
            jax.ShapeDtypeStruct((n, 2 * d), jnp.float32),
            jax.ShapeDtypeStruct((n, 2 * d), jnp.float32),
            jax.ShapeDtypeStruct((n, 2), jnp.float32),
        ],
        compiler_params=pltpu.CompilerParams(
            dimension_semantics=("arbitrary",)),
    )(emb, ws, wd, wl, nb, wa)


_C = 32


def _sc_gather(tsrc, tdst, i0, i1, pas, pad):
    e = i0.shape[0]
    n, d2 = tsrc.shape
    mesh = plsc.VectorSubcoreMesh(core_axis_name="c", subcore_axis_name="s")
    nw = 32
    chunks = e // _C

    cp = pltpu.CompilerParams()
    if "needs_layout_passes" in pltpu.CompilerParams.__dataclass_fields__:
        cp = dataclasses.replace(cp, needs_layout_passes=False)

    per = -(-(e // nw) // _C) * _C
    tail = e - (nw - 1) * per
    assert tail > 0 and tail % _C == 0 and per % 8 == 0

    buf_t = pltpu.VMEM((_C, d2), jnp.float32)
    ga_t = pltpu.VMEM((_C,), jnp.float32)

    @functools.partial(
        pl.kernel, mesh=mesh,
        compiler_params=cp,
        out_type=(jax.ShapeDtypeStruct((e, d2), jnp.float32),
                  jax.ShapeDtypeStruct((e,), jnp.float32)),
        scratch_types=[
            pltpu.VMEM((per,), jnp.int32), pltpu.VMEM((per,), jnp.int32),
            buf_t, buf_t, buf_t, buf_t,
            ga_t, ga_t,
            pltpu.VMEM((n,), jnp.float32), pltpu.VMEM((n,), jnp.float32),
            pltpu.SemaphoreType.DMA, pltpu.SemaphoreType.DMA,
            pltpu.SemaphoreType.DMA, pltpu.SemaphoreType.DMA,
        ],
    )
    def fn(tsrc_hbm, tdst_hbm, i0_hbm, i1_hbm, pas_hbm, pad_hbm,
           g_out, ga_out,
           idx0_v, idx1_v, bsa, bda, bsb, bdb, gaa, gab,
           pas_v, pad_v, sema, semb, semwa, semwb):
        cid = lax.axis_index("c")
        sid = lax.axis_index("s")
        wid = sid * 2 + cid
        base_w = wid * per
        nch = jnp.where(wid == nw - 1, tail // _C, per // _C)
        pltpu.sync_copy(pas_hbm, pas_v)
        pltpu.sync_copy(pad_hbm, pad_v)

        @pl.when(wid < nw - 1)
        def _():
            pltpu.sync_copy(i0_hbm.at[pl.ds(base_w, per)], idx0_v)
            pltpu.sync_copy(i1_hbm.at[pl.ds(base_w, per)], idx1_v)

        @pl.when(wid == nw - 1)
        def _():
            pltpu.sync_copy(i0_hbm.at[pl.ds(base_w, tail)],
                            idx0_v.at[pl.ds(0, tail)])
            pltpu.sync_copy(i1_hbm.at[pl.ds(base_w, tail)],
                            idx1_v.at[pl.ds(0, tail)])

        def drain(bufs, gabuf, semw):
            pltpu.make_async_copy(bufs, g_out.at[pl.ds(0, _C)], semw).wait()
            pltpu.make_async_copy(gabuf, ga_out.at[pl.ds(0, _C)], semw).wait()

        def issue(t, bufs, bufd, sem):
            off = t * _C
            return (
                pltpu.async_copy(tsrc_hbm.at[idx0_v.at[pl.ds(off, _C)]],
                                 bufs, sem),
                pltpu.async_copy(tdst_hbm.at[idx1_v.at[pl.ds(off, _C)]],
                                 bufd, sem),
            )

        def wait_issue(bufs, bufd, sem):
            pltpu.make_async_copy(tsrc_hbm.at[idx0_v.at[pl.ds(0, _C)]],
                                  bufs, sem).wait()
            pltpu.make_async_copy(tdst_hbm.at[idx1_v.at[pl.ds(0, _C)]],
                                  bufd, sem).wait()

        def ga_compute(t, gabuf):
            off = t * _C

            @pl.loop(0, _C, step=16)
            def _ga(k):
                v0 = idx0_v[pl.ds(off + k, 16)]
                v1 = idx1_v[pl.ds(off + k, 16)]
                gabuf[pl.ds(k, 16)] = (plsc.load_gather(pas_v, [v0]) +
                                       plsc.load_gather(pad_v, [v1]))

        def add_rows(bufs, bufd):
            @pl.loop(0, _C)
            def _row(r):
                for j in range(d2 // 16):
                    sl = pl.ds(j * 16, 16)
                    bufs[r, sl] = bufs[r, sl] + bufd[r, sl]

        def writeback(t, bufs, gabuf, semw):
            base = base_w + t * _C
            pltpu.async_copy(bufs, g_out.at[pl.ds(base, _C)], semw)
            pltpu.async_copy(gabuf, ga_out.at[pl.ds(base, _C)], semw)

        @pl.loop(0, per // _C, step=2)
        def _pair(t0):
            t1 = t0 + 1

            @pl.when(t0 < nch)
            def _():
                @pl.when(t0 > 0)
                def _():
                    drain(bsa, gaa, semwa)
                issue(t0, bsa, bda, sema)

                @pl.when(t1 < nch)
                def _():
                    @pl.when(t0 > 0)
                    def _():
                        drain(bsb, gab, semwb)
                    issue(t1, bsb, bdb, semb)

                ga_compute(t0, gaa)
                wait_issue(bsa, bda, sema)
                add_rows(bsa, bda)
                writeback(t0, bsa, gaa, semwa)

                @pl.when(t1 < nch)
                def _():
                    ga_compute(t1, gab)
                    wait_issue(bsb, bdb, semb)
                    add_rows(bsb, bdb)
                    writeback(t1, bsb, gab, semwb)

        drain(bsa, gaa, semwa)

        @pl.when(nch >= 2)
        def _():
            drain(bsb, gab, semwb)

    return fn(tsrc, tdst, i0, i1, pas, pad)


def _make_edge1_body(has_prev):
    def body(*refs):
        if has_prev:
            (x_ref, g_ref, ga_ref, we_ref, wae_ref, ub_ref, ab_ref,
             w2_ref, lb_ref, _xsp_ref, _mrp_ref, mp_ref, zp_ref,
             xo_ref, mrow_ref, m_ref, z_ref) = refs
        else:
            (x_ref, g_ref, ga_ref, we_ref, wae_ref, ub_ref, ab_ref,
             w2_ref, lb_ref, xo_ref, mrow_ref, m_ref, z_ref) = refs
        pid = pl.program_id(0)
        d = x_ref.shape[1]
        x = x_ref[...]
        g = g_ref[...]
        eu = jnp.dot(x, we_ref[...], preferred_element_type=jnp.float32)
        eu = eu + ub_ref[...]
        ea = jnp.dot(x, wae_ref[...], preferred_element_type=jnp.float32)
        pre = ea + ab_ref[...] + ga_ref[...]
        pre = jnp.where(pre >= 0, pre, 0.2 * pre)
        a = jax.nn.sigmoid(pre)
        upd = (g[:, :d] + eu) * a
        ef = jnp.dot(upd, w2_ref[...], preferred_element_type=jnp.float32)
        s = ef + lb_ref[...] + g[:, d:]
        s = jnp.where(s >= 0, s, 0.2 * s)

        @pl.when(pid == 0)
        def _():
            if has_prev:
                m_ref[...] = mp_ref[...]
                z_ref[...] = zp_ref[...]
            else:
                m_ref[...] = jnp.full(m_ref.shape, -jnp.inf, jnp.float32)
                z_ref[...] = jnp.zeros(z_ref.shape, jnp.float32)

        m_old = m_ref[...]
        m_new = jnp.maximum(m_old, jnp.max(s, axis=0, keepdims=True))
        expv = jnp.exp(s - m_new)
        z_new = z_ref[...] * jnp.exp(m_old - m_new) + jnp.sum(
            expv, axis=0, keepdims=True)
        m_ref[...] = m_new
        z_ref[...] = z_new
        xo_ref[...] = upd * expv
        mrow_ref[...] = m_new.reshape(mrow_ref.shape)

    return body


def _edge_pass1(x, g, ga, we, wae, ub, ab, w2, lb, be, off, nb_total,
                prev=None):
    eh = g.shape[0]
    d = x.shape[1]
    e = x.shape[0]
    grid = (eh // be,)
    full = lambda shape: pl.BlockSpec(shape, lambda i: (0, 0))
    in_specs = [
        pl.BlockSpec((be, d), lambda i: (i + off, 0)),
        pl.BlockSpec((be, 2 * d), lambda i: (i, 0)),
        pl.BlockSpec((be, 1), lambda i: (i, 0)),
        full((d, d)), full((d, 1)), full((1, d)), full((1, 1)),
        full((d, d)), full((1, d)),
    ]
    operands = [x, g, ga, we, wae, ub, ab, w2, lb]
    aliases = {}
    if prev is not None:
        xsp, mrp, mp, zp = prev
        in_specs += [
            pl.BlockSpec((8, d), lambda i: (0, 0)),
            pl.BlockSpec((1, 1, d), lambda i: (0, 0, 0)),
            full((1, d)), full((1, d)),
        ]
        operands += [xsp, mrp, mp, zp]
        aliases = {9: 0, 10: 1}
    return pl.pallas_call(
        _make_edge1_body(prev is not None),
        grid=grid,
        in_specs=in_specs,
        out_specs=[
            pl.BlockSpec((be, d), lambda i: (i + off, 0)),
            pl.BlockSpec((1, 1, d), lambda i: (i + off, 0, 0)),
            pl.BlockSpec((1, d), lambda i: (0, 0)),
            pl.BlockSpec((1, d), lambda i: (0, 0)),
        ],
        out_shape=[
            jax.ShapeDtypeStruct((e, d), jnp.float32),
            jax.ShapeDtypeStruct((nb_total, 1, d), jnp.float32),
            jax.ShapeDtypeStruct((1, d), jnp.float32),
            jax.ShapeDtypeStruct((1, d), jnp.float32),
        ],
        input_output_aliases=aliases,
        compiler_params=pltpu.CompilerParams(
            dimension_semantics=("arbitrary",)),
    )(*operands)


def _edge2_body(x_ref, mrow_ref, m_ref, z_ref, o_ref):
    scale = jnp.exp(mrow_ref[0] - m_ref[...]) / z_ref[...]
    o_ref[...] = x_ref[...] * scale


def _edge_pass2(xs, mrow, m, z, be):
    e, d = xs.shape
    grid = (e // be,)
    return pl.pallas_call(
        _edge2_body,
        grid=grid,
        in_specs=[
            pl.BlockSpec((be, d), lambda i: (i, 0)),
            pl.BlockSpec((1, 1, d), lambda i: (i, 0, 0)),
            pl.BlockSpec((1, d), lambda i: (0, 0)),
            pl.BlockSpec((1, d), lambda i: (0, 0)),
        ],
        out_specs=pl.BlockSpec((be, d), lambda i: (i, 0)),
        out_shape=jax.ShapeDtypeStruct((e, d), jnp.float32),
        compiler_params=pltpu.CompilerParams(
            dimension_semantics=("arbitrary",)),
    )(xs, mrow, m, z)


def kernel(edge_embeddings, edge_index, edge_attr, node_embeddings,
           num_nodes, attn_W, attn_b, update_W, update_b,
           edge_lin_W, edge_lin_b, node_lin_W, node_lin_b):
    del edge_attr, num_nodes
    e, d = edge_embeddings.shape
    n = node_embeddings.shape[0]

    i0 = edge_index[0].astype(jnp.int32)
    i1 = edge_index[1].astype(jnp.int32)

    ws, wd, we = update_W[:d], update_W[d:2 * d], update_W[2 * d:]
    wa2 = jnp.concatenate([attn_W[:d], attn_W[d:2 * d]], axis=1)
    wae = attn_W[2 * d:]

    bn = 2000 if n % 2000 == 0 else n
    tsrc, tdst, p2 = _node_precompute(
        node_embeddings, ws, wd, node_lin_W,
        node_lin_b.reshape(1, d), wa2, bn)

    pas, pad = p2[:, 0], p2[:, 1]
    be = 2000 if e % 2000 == 0 else e
    nb_total = e // be
    ub = update_b.reshape(1, d)
    ab = attn_b.reshape(1, 1)
    lb = edge_lin_b.reshape(1, d)

    half = e // 2
    if half % be == 0 and half % _C == 0:
        g0, ga0 = _sc_gather(tsrc, tdst, i0[:half], i1[:half], pas, pad)
        g1, ga1 = _sc_gather(tsrc, tdst, i0[half:], i1[half:], pas, pad)
        prev = _edge_pass1(edge_embeddings, g0, ga0.reshape(half, 1), we,
                           wae, ub, ab, edge_lin_W, lb, be, 0, nb_total)
        xs, mrow, m, z = _edge_pass1(
            edge_embeddings, g1, ga1.reshape(half, 1), we, wae, ub, ab,
            edge_lin_W, lb, be, half // be, nb_total, prev=prev)
    else:
        g, ga = _sc_gather(tsrc, tdst, i0, i1, pas, pad)
        xs, mrow, m, z = _edge_pass1(
            edge_embeddings, g, ga.reshape(e, 1), we, wae, ub, ab,
            edge_lin_W, lb, be, 0, nb_total)

    return _edge_pass2(xs, mrow, m, z, be)


# speedup vs baseline: 1.9183x; 1.0057x over previous
import dataclasses
import functools

import jax
import jax.numpy as jnp
from jax import lax
from jax.experimental import pallas as pl
from jax.experimental.pallas import tpu as pltpu
from jax.experimental.pallas import tpu_sc as plsc


def _node_body(emb_ref, ws_ref, wd_ref, wl_ref, nb_ref, wa_ref,
               tsrc_ref, tdst_ref, p2_ref):
    emb = emb_ref[...]
    d = emb.shape[1]
    pl_feat = jnp.dot(emb, wl_ref[...], preferred_element_type=jnp.float32)
    pl_feat = pl_feat + nb_ref[...]
    tsrc_ref[:, :d] = jnp.dot(emb, ws_ref[...],
                              preferred_element_type=jnp.float32)
    tsrc_ref[:, d:] = pl_feat
    tdst_ref[:, :d] = jnp.dot(emb, wd_ref[...],
                              preferred_element_type=jnp.float32)
    tdst_ref[:, d:] = pl_feat
    p2_ref[...] = jnp.dot(emb, wa_ref[...], preferred_element_type=jnp.float32)


def _node_precompute(emb, ws, wd, wl, nb, wa, bn):
    n, d = emb.shape
    grid = (n // bn,)
    full = lambda shape: pl.BlockSpec(shape, lambda i: (0, 0))
    return pl.pallas_call(
        _node_body,
        grid=grid,
        in_specs=[
            pl.BlockSpec((bn, d), lambda i: (i, 0)),
            full((d, d)), full((d, d)), full((d, d)),
            full((1, d)), full((d, 2)),
        ],
        out_specs=[
            pl.BlockSpec((bn, 2 * d), lambda i: (i, 0)),
            pl.BlockSpec((bn, 2 * d), lambda i: (i, 0)),
            pl.BlockSpec((bn, 2), lambda i: (i, 0)),
        ],
        out_shape=[
            jax.ShapeDtypeStruct((n, 2 * d), jnp.float32),
            jax.ShapeDtypeStruct((n, 2 * d), jnp.float32),
            jax.ShapeDtypeStruct((n, 2), jnp.float32),
        ],
        compiler_params=pltpu.CompilerParams(
            dimension_semantics=("arbitrary",)),
    )(emb, ws, wd, wl, nb, wa)


_C = 32


def _sc_gather(tsrc, tdst, i0, i1, pas, pad):
    e = i0.shape[0]
    n, d2 = tsrc.shape
    mesh = plsc.VectorSubcoreMesh(core_axis_name="c", subcore_axis_name="s")
    nw = 32
    chunks = e // _C

    cp = pltpu.CompilerParams()
    if "needs_layout_passes" in pltpu.CompilerParams.__dataclass_fields__:
        cp = dataclasses.replace(cp, needs_layout_passes=False)

    per = -(-(e // nw) // _C) * _C
    tail = e - (nw - 1) * per
    assert tail > 0 and tail % _C == 0 and per % 8 == 0

    buf_t = pltpu.VMEM((_C, d2), jnp.float32)
    ga_t = pltpu.VMEM((_C,), jnp.float32)

    @functools.partial(
        pl.kernel, mesh=mesh,
        compiler_params=cp,
        out_type=(jax.ShapeDtypeStruct((e, d2), jnp.float32),
                  jax.ShapeDtypeStruct((e,), jnp.float32)),
        scratch_types=[
            pltpu.VMEM((per,), jnp.int32), pltpu.VMEM((per,), jnp.int32),
            buf_t, buf_t, buf_t, buf_t,
            ga_t, ga_t,
            pltpu.VMEM((n,), jnp.float32), pltpu.VMEM((n,), jnp.float32),
            pltpu.SemaphoreType.DMA, pltpu.SemaphoreType.DMA,
            pltpu.SemaphoreType.DMA, pltpu.SemaphoreType.DMA,
        ],
    )
    def fn(tsrc_hbm, tdst_hbm, i0_hbm, i1_hbm, pas_hbm, pad_hbm,
           g_out, ga_out,
           idx0_v, idx1_v, bsa, bda, bsb, bdb, gaa, gab,
           pas_v, pad_v, sema, semb, semwa, semwb):
        cid = lax.axis_index("c")
        sid = lax.axis_index("s")
        wid = sid * 2 + cid
        base_w = wid * per
        nch = jnp.where(wid == nw - 1, tail // _C, per // _C)
        pltpu.sync_copy(pas_hbm, pas_v)
        pltpu.sync_copy(pad_hbm, pad_v)

        @pl.when(wid < nw - 1)
        def _():
            pltpu.sync_copy(i0_hbm.at[pl.ds(base_w, per)], idx0_v)
            pltpu.sync_copy(i1_hbm.at[pl.ds(base_w, per)], idx1_v)

        @pl.when(wid == nw - 1)
        def _():
            pltpu.sync_copy(i0_hbm.at[pl.ds(base_w, tail)],
                            idx0_v.at[pl.ds(0, tail)])
            pltpu.sync_copy(i1_hbm.at[pl.ds(base_w, tail)],
                            idx1_v.at[pl.ds(0, tail)])

        def drain(bufs, gabuf, semw):
            pltpu.make_async_copy(bufs, g_out.at[pl.ds(0, _C)], semw).wait()
            pltpu.make_async_copy(gabuf, ga_out.at[pl.ds(0, _C)], semw).wait()

        def issue(t, bufs, bufd, sem):
            off = t * _C
            return (
                pltpu.async_copy(tsrc_hbm.at[idx0_v.at[pl.ds(off, _C)]],
                                 bufs, sem),
                pltpu.async_copy(tdst_hbm.at[idx1_v.at[pl.ds(off, _C)]],
                                 bufd, sem),
            )

        def wait_issue(bufs, bufd, sem):
            pltpu.make_async_copy(tsrc_hbm.at[idx0_v.at[pl.ds(0, _C)]],
                                  bufs, sem).wait()
            pltpu.make_async_copy(tdst_hbm.at[idx1_v.at[pl.ds(0, _C)]],
                                  bufd, sem).wait()

        def ga_compute(t, gabuf):
            off = t * _C

            @pl.loop(0, _C, step=16)
            def _ga(k):
                v0 = idx0_v[pl.ds(off + k, 16)]
                v1 = idx1_v[pl.ds(off + k, 16)]
                gabuf[pl.ds(k, 16)] = (plsc.load_gather(pas_v, [v0]) +
                                       plsc.load_gather(pad_v, [v1]))

        def add_rows(bufs, bufd):
            @pl.loop(0, _C)
            def _row(r):
                for j in range(d2 // 16):
                    sl = pl.ds(j * 16, 16)
                    plsc.addupdate(bufs.at[r, sl], bufd[r, sl])

        def writeback(t, bufs, gabuf, semw):
            base = base_w + t * _C
            pltpu.async_copy(bufs, g_out.at[pl.ds(base, _C)], semw)
            pltpu.async_copy(gabuf, ga_out.at[pl.ds(base, _C)], semw)

        @pl.loop(0, per // _C, step=2)
        def _pair(t0):
            t1 = t0 + 1

            @pl.when(t0 < nch)
            def _():
                @pl.when(t0 > 0)
                def _():
                    drain(bsa, gaa, semwa)
                issue(t0, bsa, bda, sema)

                @pl.when(t1 < nch)
                def _():
                    @pl.when(t0 > 0)
                    def _():
                        drain(bsb, gab, semwb)
                    issue(t1, bsb, bdb, semb)

                ga_compute(t0, gaa)
                wait_issue(bsa, bda, sema)
                add_rows(bsa, bda)
                writeback(t0, bsa, gaa, semwa)

                @pl.when(t1 < nch)
                def _():
                    ga_compute(t1, gab)
                    wait_issue(bsb, bdb, semb)
                    add_rows(bsb, bdb)
                    writeback(t1, bsb, gab, semwb)

        drain(bsa, gaa, semwa)

        @pl.when(nch >= 2)
        def _():
            drain(bsb, gab, semwb)

    return fn(tsrc, tdst, i0, i1, pas, pad)


def _make_edge1_body(has_prev):
    def body(*refs):
        if has_prev:
            (x_ref, g_ref, ga_ref, we_ref, wae_ref, ub_ref, ab_ref,
             w2_ref, lb_ref, _xsp_ref, _mrp_ref, mp_ref, zp_ref,
             xo_ref, mrow_ref, m_ref, z_ref) = refs
        else:
            (x_ref, g_ref, ga_ref, we_ref, wae_ref, ub_ref, ab_ref,
             w2_ref, lb_ref, xo_ref, mrow_ref, m_ref, z_ref) = refs
        pid = pl.program_id(0)
        d = x_ref.shape[1]
        x = x_ref[...]
        g = g_ref[...]
        eu = jnp.dot(x.astype(jnp.bfloat16),
                     we_ref[...].astype(jnp.bfloat16),
                     preferred_element_type=jnp.float32)
        eu = eu + ub_ref[...]
        ea = jnp.dot(x, wae_ref[...], preferred_element_type=jnp.float32)
        pre = ea + ab_ref[...] + ga_ref[...]
        pre = jnp.where(pre >= 0, pre, 0.2 * pre)
        a = jax.nn.sigmoid(pre)
        upd = (g[:, :d] + eu) * a
        ef = jnp.dot(upd, w2_ref[...], preferred_element_type=jnp.float32)
        s = ef + lb_ref[...] + g[:, d:]
        s = jnp.where(s >= 0, s, 0.2 * s)

        @pl.when(pid == 0)
        def _():
            if has_prev:
                m_ref[...] = mp_ref[...]
                z_ref[...] = zp_ref[...]
            else:
                m_ref[...] = jnp.full(m_ref.shape, -jnp.inf, jnp.float32)
                z_ref[...] = jnp.zeros(z_ref.shape, jnp.float32)

        m_old = m_ref[...]
        m_new = jnp.maximum(m_old, jnp.max(s, axis=0, keepdims=True))
        expv = jnp.exp(s - m_new)
        z_new = z_ref[...] * jnp.exp(m_old - m_new) + jnp.sum(
            expv, axis=0, keepdims=True)
        m_ref[...] = m_new
        z_ref[...] = z_new
        xo_ref[...] = upd * expv
        mrow_ref[...] = m_new.reshape(mrow_ref.shape)

    return body


def _edge_pass1(x, g, ga, we, wae, ub, ab, w2, lb, be, off, nb_total,
                prev=None):
    eh = g.shape[0]
    d = x.shape[1]
    e = x.shape[0]
    grid = (eh // be,)
    full = lambda shape: pl.BlockSpec(shape, lambda i: (0, 0))
    in_specs = [
        pl.BlockSpec((be, d), lambda i: (i + off, 0)),
        pl.BlockSpec((be, 2 * d), lambda i: (i, 0)),
        pl.BlockSpec((be, 1), lambda i: (i, 0)),
        full((d, d)), full((d, 1)), full((1, d)), full((1, 1)),
        full((d, d)), full((1, d)),
    ]
    operands = [x, g, ga, we, wae, ub, ab, w2, lb]
    aliases = {}
    if prev is not None:
        xsp, mrp, mp, zp = prev
        in_specs += [
            pl.BlockSpec((8, d), lambda i: (0, 0)),
            pl.BlockSpec((1, 1, d), lambda i: (0, 0, 0)),
            full((1, d)), full((1, d)),
        ]
        operands += [xsp, mrp, mp, zp]
        aliases = {9: 0, 10: 1}
    return pl.pallas_call(
        _make_edge1_body(prev is not None),
        grid=grid,
        in_specs=in_specs,
        out_specs=[
            pl.BlockSpec((be, d), lambda i: (i + off, 0)),
            pl.BlockSpec((1, 1, d), lambda i: (i + off, 0, 0)),
            pl.BlockSpec((1, d), lambda i: (0, 0)),
            pl.BlockSpec((1, d), lambda i: (0, 0)),
        ],
        out_shape=[
            jax.ShapeDtypeStruct((e, d), jnp.float32),
            jax.ShapeDtypeStruct((nb_total, 1, d), jnp.float32),
            jax.ShapeDtypeStruct((1, d), jnp.float32),
            jax.ShapeDtypeStruct((1, d), jnp.float32),
        ],
        input_output_aliases=aliases,
        compiler_params=pltpu.CompilerParams(
            dimension_semantics=("arbitrary",)),
    )(*operands)


def _edge2_body(x_ref, mrow_ref, m_ref, z_ref, o_ref):
    scale = jnp.exp(mrow_ref[0] - m_ref[...]) / z_ref[...]
    o_ref[...] = x_ref[...] * scale


def _edge_pass2(xs, mrow, m, z, be):
    e, d = xs.shape
    grid = (e // be,)
    return pl.pallas_call(
        _edge2_body,
        grid=grid,
        in_specs=[
            pl.BlockSpec((be, d), lambda i: (i, 0)),
            pl.BlockSpec((1, 1, d), lambda i: (i, 0, 0)),
            pl.BlockSpec((1, d), lambda i: (0, 0)),
            pl.BlockSpec((1, d), lambda i: (0, 0)),
        ],
        out_specs=pl.BlockSpec((be, d), lambda i: (i, 0)),
        out_shape=jax.ShapeDtypeStruct((e, d), jnp.float32),
        compiler_params=pltpu.CompilerParams(
            dimension_semantics=("arbitrary",)),
    )(xs, mrow, m, z)


def kernel(edge_embeddings, edge_index, edge_attr, node_embeddings,
           num_nodes, attn_W, attn_b, update_W, update_b,
           edge_lin_W, edge_lin_b, node_lin_W, node_lin_b):
    del edge_attr, num_nodes
    e, d = edge_embeddings.shape
    n = node_embeddings.shape[0]

    i0 = edge_index[0].astype(jnp.int32)
    i1 = edge_index[1].astype(jnp.int32)

    ws, wd, we = update_W[:d], update_W[d:2 * d], update_W[2 * d:]
    wa2 = jnp.concatenate([attn_W[:d], attn_W[d:2 * d]], axis=1)
    wae = attn_W[2 * d:]

    bn = 2000 if n % 2000 == 0 else n
    tsrc, tdst, p2 = _node_precompute(
        node_embeddings, ws, wd, node_lin_W,
        node_lin_b.reshape(1, d), wa2, bn)

    pas, pad = p2[:, 0], p2[:, 1]
    be = 2000 if e % 2000 == 0 else e
    nb_total = e // be
    ub = update_b.reshape(1, d)
    ab = attn_b.reshape(1, 1)
    lb = edge_lin_b.reshape(1, d)

    half = e // 2
    if half % be == 0 and half % _C == 0:
        g0, ga0 = _sc_gather(tsrc, tdst, i0[:half], i1[:half], pas, pad)
        g1, ga1 = _sc_gather(tsrc, tdst, i0[half:], i1[half:], pas, pad)
        prev = _edge_pass1(edge_embeddings, g0, ga0.reshape(half, 1), we,
                           wae, ub, ab, edge_lin_W, lb, be, 0, nb_total)
        xs, mrow, m, z = _edge_pass1(
            edge_embeddings, g1, ga1.reshape(half, 1), we, wae, ub, ab,
            edge_lin_W, lb, be, half // be, nb_total, prev=prev)
    else:
        g, ga = _sc_gather(tsrc, tdst, i0, i1, pas, pad)
        xs, mrow, m, z = _edge_pass1(
            edge_embeddings, g, ga.reshape(e, 1), we, wae, ub, ab,
            edge_lin_W, lb, be, 0, nb_total)

    return _edge_pass2(xs, mrow, m, z, be)


# speedup vs baseline: 2.5365x; 1.3222x over previous
import dataclasses
import functools

import jax
import jax.numpy as jnp
from jax import lax
from jax.experimental import pallas as pl
from jax.experimental.pallas import tpu as pltpu
from jax.experimental.pallas import tpu_sc as plsc


def _pack16(u, l):
    ub = lax.bitcast_convert_type(u.astype(jnp.bfloat16).astype(jnp.float32),
                                  jnp.int32)
    lb = lax.bitcast_convert_type(l.astype(jnp.bfloat16).astype(jnp.float32),
                                  jnp.int32)
    return ((ub >> 16) & jnp.int32(0xFFFF)) | (lb & jnp.int32(-65536))


def _node_body(emb_ref, ws_ref, wd_ref, wl_ref, nb_ref, wa_ref,
               tsrc_ref, tdst_ref, p2_ref):
    emb = emb_ref[...]
    pl_feat = jnp.dot(emb, wl_ref[...], preferred_element_type=jnp.float32)
    pl_feat = pl_feat + nb_ref[...]
    ps = jnp.dot(emb, ws_ref[...], preferred_element_type=jnp.float32)
    pd = jnp.dot(emb, wd_ref[...], preferred_element_type=jnp.float32)
    tsrc_ref[...] = _pack16(ps, pl_feat)
    tdst_ref[...] = _pack16(pd, pl_feat)
    p2_ref[...] = jnp.dot(emb, wa_ref[...], preferred_element_type=jnp.float32)


def _node_precompute(emb, ws, wd, wl, nb, wa, bn):
    n, d = emb.shape
    grid = (n // bn,)
    full = lambda shape: pl.BlockSpec(shape, lambda i: (0, 0))
    return pl.pallas_call(
        _node_body,
        grid=grid,
        in_specs=[
            pl.BlockSpec((bn, d), lambda i: (i, 0)),
            full((d, d)), full((d, d)), full((d, d)),
            full((1, d)), full((d, 2)),
        ],
        out_specs=[
            pl.BlockSpec((bn, d), lambda i: (i, 0)),
            pl.BlockSpec((bn, d), lambda i: (i, 0)),
            pl.BlockSpec((bn, 2), lambda i: (i, 0)),
        ],
        out_shape=[
            jax.ShapeDtypeStruct((n, d), jnp.int32),
            jax.ShapeDtypeStruct((n, d), jnp.int32),
            jax.ShapeDtypeStruct((n, 2), jnp.float32),
        ],
        compiler_params=pltpu.CompilerParams(
            dimension_semantics=("arbitrary",)),
    )(emb, ws, wd, wl, nb, wa)


_C = 32


def _sc_gather(tsrc, tdst, i0, i1, pas, pad):
    e = i0.shape[0]
    n, d2 = tsrc.shape
    mesh = plsc.VectorSubcoreMesh(core_axis_name="c", subcore_axis_name="s")
    nw = 32
    chunks = e // _C

    cp = pltpu.CompilerParams()
    if "needs_layout_passes" in pltpu.CompilerParams.__dataclass_fields__:
        cp = dataclasses.replace(cp, needs_layout_passes=False)

    per = -(-(e // nw) // _C) * _C
    tail = e - (nw - 1) * per
    assert tail > 0 and tail % _C == 0 and per % 8 == 0

    buf_t = pltpu.VMEM((_C, d2), jnp.int32)
    ga_t = pltpu.VMEM((_C,), jnp.float32)

    @functools.partial(
        pl.kernel, mesh=mesh,
        compiler_params=cp,
        out_type=(jax.ShapeDtypeStruct((e, d2), jnp.int32),
                  jax.ShapeDtypeStruct((e,), jnp.float32)),
        scratch_types=[
            pltpu.VMEM((per,), jnp.int32), pltpu.VMEM((per,), jnp.int32),
            buf_t, buf_t, buf_t, buf_t,
            ga_t, ga_t,
            pltpu.VMEM((n,), jnp.float32), pltpu.VMEM((n,), jnp.float32),
            pltpu.SemaphoreType.DMA, pltpu.SemaphoreType.DMA,
            pltpu.SemaphoreType.DMA, pltpu.SemaphoreType.DMA,
        ],
    )
    def fn(tsrc_hbm, tdst_hbm, i0_hbm, i1_hbm, pas_hbm, pad_hbm,
           g_out, ga_out,
           idx0_v, idx1_v, bsa, bda, bsb, bdb, gaa, gab,
           pas_v, pad_v, sema, semb, semwa, semwb):
        cid = lax.axis_index("c")
        sid = lax.axis_index("s")
        wid = sid * 2 + cid
        base_w = wid * per
        nch = jnp.where(wid == nw - 1, tail // _C, per // _C)
        pltpu.sync_copy(pas_hbm, pas_v)
        pltpu.sync_copy(pad_hbm, pad_v)

        @pl.when(wid < nw - 1)
        def _():
            pltpu.sync_copy(i0_hbm.at[pl.ds(base_w, per)], idx0_v)
            pltpu.sync_copy(i1_hbm.at[pl.ds(base_w, per)], idx1_v)

        @pl.when(wid == nw - 1)
        def _():
            pltpu.sync_copy(i0_hbm.at[pl.ds(base_w, tail)],
                            idx0_v.at[pl.ds(0, tail)])
            pltpu.sync_copy(i1_hbm.at[pl.ds(base_w, tail)],
                            idx1_v.at[pl.ds(0, tail)])

        def drain(bufs, gabuf, semw):
            pltpu.make_async_copy(bufs, g_out.at[pl.ds(0, _C)], semw).wait()
            pltpu.make_async_copy(gabuf, ga_out.at[pl.ds(0, _C)], semw).wait()

        def issue(t, bufs, bufd, sem):
            off = t * _C
            return (
                pltpu.async_copy(tsrc_hbm.at[idx0_v.at[pl.ds(off, _C)]],
                                 bufs, sem),
                pltpu.async_copy(tdst_hbm.at[idx1_v.at[pl.ds(off, _C)]],
                                 bufd, sem),
            )

        def wait_issue(bufs, bufd, sem):
            pltpu.make_async_copy(tsrc_hbm.at[idx0_v.at[pl.ds(0, _C)]],
                                  bufs, sem).wait()
            pltpu.make_async_copy(tdst_hbm.at[idx1_v.at[pl.ds(0, _C)]],
                                  bufd, sem).wait()

        def ga_compute(t, gabuf):
            off = t * _C

            @pl.loop(0, _C, step=16)
            def _ga(k):
                v0 = idx0_v[pl.ds(off + k, 16)]
                v1 = idx1_v[pl.ds(off + k, 16)]
                gabuf[pl.ds(k, 16)] = (plsc.load_gather(pas_v, [v0]) +
                                       plsc.load_gather(pad_v, [v1]))

        def add_rows(bufs, bufd):
            @pl.loop(0, _C)
            def _row(r):
                for j in range(d2 // 16):
                    cut = pl.ds(j * 16, 16)
                    va = plsc.bitcast(bufs[r, cut], jnp.bfloat16)
                    vb = plsc.bitcast(bufd[r, cut], jnp.bfloat16)
                    bufs[r, cut] = plsc.bitcast(va + vb, jnp.int32)

        def writeback(t, bufs, gabuf, semw):
            base = base_w + t * _C
            pltpu.async_copy(bufs, g_out.at[pl.ds(base, _C)], semw)
            pltpu.async_copy(gabuf, ga_out.at[pl.ds(base, _C)], semw)

        @pl.loop(0, per // _C, step=2)
        def _pair(t0):
            t1 = t0 + 1

            @pl.when(t0 < nch)
            def _():
                @pl.when(t0 > 0)
                def _():
                    drain(bsa, gaa, semwa)
                issue(t0, bsa, bda, sema)

                @pl.when(t1 < nch)
                def _():
                    @pl.when(t0 > 0)
                    def _():
                        drain(bsb, gab, semwb)
                    issue(t1, bsb, bdb, semb)

                ga_compute(t0, gaa)
                wait_issue(bsa, bda, sema)
                add_rows(bsa, bda)
                writeback(t0, bsa, gaa, semwa)

                @pl.when(t1 < nch)
                def _():
                    ga_compute(t1, gab)
                    wait_issue(bsb, bdb, semb)
                    add_rows(bsb, bdb)
                    writeback(t1, bsb, gab, semwb)

        drain(bsa, gaa, semwa)

        @pl.when(nch >= 2)
        def _():
            drain(bsb, gab, semwb)

    return fn(tsrc, tdst, i0, i1, pas, pad)


def _make_edge1_body(has_prev):
    def body(*refs):
        if has_prev:
            (x_ref, g_ref, ga_ref, we_ref, wae_ref, ub_ref, ab_ref,
             w2_ref, lb_ref, _xsp_ref, _mrp_ref, mp_ref, zp_ref,
             xo_ref, mrow_ref, m_ref, z_ref) = refs
        else:
            (x_ref, g_ref, ga_ref, we_ref, wae_ref, ub_ref, ab_ref,
             w2_ref, lb_ref, xo_ref, mrow_ref, m_ref, z_ref) = refs
        pid = pl.program_id(0)
        d = x_ref.shape[1]
        x = x_ref[...]
        gi = g_ref[...]
        gu = lax.bitcast_convert_type(gi << 16, jnp.float32)
        gl = lax.bitcast_convert_type(gi & jnp.int32(-65536), jnp.float32)
        eu = jnp.dot(x.astype(jnp.bfloat16),
                     we_ref[...].astype(jnp.bfloat16),
                     preferred_element_type=jnp.float32)
        eu = eu + ub_ref[...]
        ea = jnp.dot(x, wae_ref[...], preferred_element_type=jnp.float32)
        pre = ea + ab_ref[...] + ga_ref[...]
        pre = jnp.where(pre >= 0, pre, 0.2 * pre)
        a = jax.nn.sigmoid(pre)
        upd = (gu + eu) * a
        ef = jnp.dot(upd, w2_ref[...], preferred_element_type=jnp.float32)
        s = ef + lb_ref[...] + gl
        s = jnp.where(s >= 0, s, 0.2 * s)

        @pl.when(pid == 0)
        def _():
            if has_prev:
                m_ref[...] = mp_ref[...]
                z_ref[...] = zp_ref[...]
            else:
                m_ref[...] = jnp.full(m_ref.shape, -jnp.inf, jnp.float32)
                z_ref[...] = jnp.zeros(z_ref.shape, jnp.float32)

        m_old = m_ref[...]
        m_new = jnp.maximum(m_old, jnp.max(s, axis=0, keepdims=True))
        expv = jnp.exp(s - m_new)
        z_new = z_ref[...] * jnp.exp(m_old - m_new) + jnp.sum(
            expv, axis=0, keepdims=True)
        m_ref[...] = m_new
        z_ref[...] = z_new
        xo_ref[...] = upd * expv
        mrow_ref[...] = m_new.reshape(mrow_ref.shape)

    return body


def _edge_pass1(x, g, ga, we, wae, ub, ab, w2, lb, be, off, nb_total,
                prev=None):
    eh = g.shape[0]
    d = x.shape[1]
    e = x.shape[0]
    grid = (eh // be,)
    full = lambda shape: pl.BlockSpec(shape, lambda i: (0, 0))
    in_specs = [
        pl.BlockSpec((be, d), lambda i: (i + off, 0)),
        pl.BlockSpec((be, d), lambda i: (i, 0)),
        pl.BlockSpec((be, 1), lambda i: (i, 0)),
        full((d, d)), full((d, 1)), full((1, d)), full((1, 1)),
        full((d, d)), full((1, d)),
    ]
    operands = [x, g, ga, we, wae, ub, ab, w2, lb]
    aliases = {}
    if prev is not None:
        xsp, mrp, mp, zp = prev
        in_specs += [
            pl.BlockSpec((8, d), lambda i: (0, 0)),
            pl.BlockSpec((1, 1, d), lambda i: (0, 0, 0)),
            full((1, d)), full((1, d)),
        ]
        operands += [xsp, mrp, mp, zp]
        aliases = {9: 0, 10: 1}
    return pl.pallas_call(
        _make_edge1_body(prev is not None),
        grid=grid,
        in_specs=in_specs,
        out_specs=[
            pl.BlockSpec((be, d), lambda i: (i + off, 0)),
            pl.BlockSpec((1, 1, d), lambda i: (i + off, 0, 0)),
            pl.BlockSpec((1, d), lambda i: (0, 0)),
            pl.BlockSpec((1, d), lambda i: (0, 0)),
        ],
        out_shape=[
            jax.ShapeDtypeStruct((e, d), jnp.float32),
            jax.ShapeDtypeStruct((nb_total, 1, d), jnp.float32),
            jax.ShapeDtypeStruct((1, d), jnp.float32),
            jax.ShapeDtypeStruct((1, d), jnp.float32),
        ],
        input_output_aliases=aliases,
        compiler_params=pltpu.CompilerParams(
            dimension_semantics=("arbitrary",)),
    )(*operands)


def _edge2_body(x_ref, mrow_ref, m_ref, z_ref, o_ref):
    scale = jnp.exp(mrow_ref[0] - m_ref[...]) / z_ref[...]
    o_ref[...] = x_ref[...] * scale


def _edge_pass2(xs, mrow, m, z, be):
    e, d = xs.shape
    grid = (e // be,)
    return pl.pallas_call(
        _edge2_body,
        grid=grid,
        in_specs=[
            pl.BlockSpec((be, d), lambda i: (i, 0)),
            pl.BlockSpec((1, 1, d), lambda i: (i, 0, 0)),
            pl.BlockSpec((1, d), lambda i: (0, 0)),
            pl.BlockSpec((1, d), lambda i: (0, 0)),
        ],
        out_specs=pl.BlockSpec((be, d), lambda i: (i, 0)),
        out_shape=jax.ShapeDtypeStruct((e, d), jnp.float32),
        compiler_params=pltpu.CompilerParams(
            dimension_semantics=("arbitrary",)),
    )(xs, mrow, m, z)


def kernel(edge_embeddings, edge_index, edge_attr, node_embeddings,
           num_nodes, attn_W, attn_b, update_W, update_b,
           edge_lin_W, edge_lin_b, node_lin_W, node_lin_b):
    del edge_attr, num_nodes
    e, d = edge_embeddings.shape
    n = node_embeddings.shape[0]

    i0 = edge_index[0].astype(jnp.int32)
    i1 = edge_index[1].astype(jnp.int32)

    ws, wd, we = update_W[:d], update_W[d:2 * d], update_W[2 * d:]
    wa2 = jnp.concatenate([attn_W[:d], attn_W[d:2 * d]], axis=1)
    wae = attn_W[2 * d:]

    bn = 2000 if n % 2000 == 0 else n
    tsrc, tdst, p2 = _node_precompute(
        node_embeddings, ws, wd, node_lin_W,
        node_lin_b.reshape(1, d), wa2, bn)

    pas, pad = p2[:, 0], p2[:, 1]
    be = 2000 if e % 2000 == 0 else e
    nb_total = e // be
    ub = update_b.reshape(1, d)
    ab = attn_b.reshape(1, 1)
    lb = edge_lin_b.reshape(1, d)

    half = e // 2
    if half % be == 0 and half % _C == 0:
        g0, ga0 = _sc_gather(tsrc, tdst, i0[:half], i1[:half], pas, pad)
        g1, ga1 = _sc_gather(tsrc, tdst, i0[half:], i1[half:], pas, pad)
        prev = _edge_pass1(edge_embeddings, g0, ga0.reshape(half, 1), we,
                           wae, ub, ab, edge_lin_W, lb, be, 0, nb_total)
        xs, mrow, m, z = _edge_pass1(
            edge_embeddings, g1, ga1.reshape(half, 1), we, wae, ub, ab,
            edge_lin_W, lb, be, half // be, nb_total, prev=prev)
    else:
        g, ga = _sc_gather(tsrc, tdst, i0, i1, pas, pad)
        xs, mrow, m, z = _edge_pass1(
            edge_embeddings, g, ga.reshape(e, 1), we, wae, ub, ab,
            edge_lin_W, lb, be, 0, nb_total)

    return _edge_pass2(xs, mrow, m, z, be)


# speedup vs baseline: 2.6423x; 1.0417x over previous
import dataclasses
import functools

import jax
import jax.numpy as jnp
from jax import lax
from jax.experimental import pallas as pl
from jax.experimental.pallas import tpu as pltpu
from jax.experimental.pallas import tpu_sc as plsc


def _pack16(u, l):
    ub = lax.bitcast_convert_type(u.astype(jnp.bfloat16).astype(jnp.float32),
                                  jnp.int32)
    lb = lax.bitcast_convert_type(l.astype(jnp.bfloat16).astype(jnp.float32),
                                  jnp.int32)
    return ((ub >> 16) & jnp.int32(0xFFFF)) | (lb & jnp.int32(-65536))


def _node_body(emb_ref, ws_ref, wd_ref, wl_ref, nb_ref, wa_ref,
               tsrc_ref, tdst_ref, p2_ref):
    emb = emb_ref[...]
    pl_feat = jnp.dot(emb, wl_ref[...], preferred_element_type=jnp.float32)
    pl_feat = pl_feat + nb_ref[...]
    ps = jnp.dot(emb, ws_ref[...], preferred_element_type=jnp.float32)
    pd = jnp.dot(emb, wd_ref[...], preferred_element_type=jnp.float32)
    tsrc_ref[...] = _pack16(ps, pl_feat)
    tdst_ref[...] = _pack16(pd, pl_feat)
    p2_ref[...] = jnp.dot(emb, wa_ref[...], preferred_element_type=jnp.float32)


def _node_precompute(emb, ws, wd, wl, nb, wa, bn):
    n, d = emb.shape
    grid = (n // bn,)
    full = lambda shape: pl.BlockSpec(shape, lambda i: (0, 0))
    return pl.pallas_call(
        _node_body,
        grid=grid,
        in_specs=[
            pl.BlockSpec((bn, d), lambda i: (i, 0)),
            full((d, d)), full((d, d)), full((d, d)),
            full((1, d)), full((d, 2)),
        ],
        out_specs=[
            pl.BlockSpec((bn, d), lambda i: (i, 0)),
            pl.BlockSpec((bn, d), lambda i: (i, 0)),
            pl.BlockSpec((bn, 2), lambda i: (i, 0)),
        ],
        out_shape=[
            jax.ShapeDtypeStruct((n, d), jnp.int32),
            jax.ShapeDtypeStruct((n, d), jnp.int32),
            jax.ShapeDtypeStruct((n, 2), jnp.float32),
        ],
        compiler_params=pltpu.CompilerParams(
            dimension_semantics=("arbitrary",)),
    )(emb, ws, wd, wl, nb, wa)


_C = 32


def _sc_gather(tsrc, tdst, i0, i1, pas, pad):
    e = i0.shape[0]
    n, d2 = tsrc.shape
    mesh = plsc.VectorSubcoreMesh(core_axis_name="c", subcore_axis_name="s")
    nw = 32
    chunks = e // _C

    cp = pltpu.CompilerParams()
    if "needs_layout_passes" in pltpu.CompilerParams.__dataclass_fields__:
        cp = dataclasses.replace(cp, needs_layout_passes=False)

    per = -(-(e // nw) // _C) * _C
    tail = e - (nw - 1) * per
    assert tail > 0 and tail % _C == 0 and per % 8 == 0

    buf_t = pltpu.VMEM((_C, d2), jnp.int32)
    ga_t = pltpu.VMEM((_C,), jnp.float32)

    @functools.partial(
        pl.kernel, mesh=mesh,
        compiler_params=cp,
        out_type=(jax.ShapeDtypeStruct((e, d2), jnp.int32),
                  jax.ShapeDtypeStruct((e,), jnp.float32)),
        scratch_types=[
            pltpu.VMEM((per,), jnp.int32), pltpu.VMEM((per,), jnp.int32),
            buf_t, buf_t, buf_t, buf_t,
            ga_t, ga_t,
            pltpu.VMEM((n,), jnp.float32), pltpu.VMEM((n,), jnp.float32),
            pltpu.SemaphoreType.DMA, pltpu.SemaphoreType.DMA,
            pltpu.SemaphoreType.DMA, pltpu.SemaphoreType.DMA,
        ],
    )
    def fn(tsrc_hbm, tdst_hbm, i0_hbm, i1_hbm, pas_hbm, pad_hbm,
           g_out, ga_out,
           idx0_v, idx1_v, bsa, bda, bsb, bdb, gaa, gab,
           pas_v, pad_v, sema, semb, semwa, semwb):
        cid = lax.axis_index("c")
        sid = lax.axis_index("s")
        wid = sid * 2 + cid
        base_w = wid * per
        nch = jnp.where(wid == nw - 1, tail // _C, per // _C)
        pltpu.sync_copy(pas_hbm, pas_v)
        pltpu.sync_copy(pad_hbm, pad_v)

        @pl.when(wid < nw - 1)
        def _():
            pltpu.sync_copy(i0_hbm.at[pl.ds(base_w, per)], idx0_v)
            pltpu.sync_copy(i1_hbm.at[pl.ds(base_w, per)], idx1_v)

        @pl.when(wid == nw - 1)
        def _():
            pltpu.sync_copy(i0_hbm.at[pl.ds(base_w, tail)],
                            idx0_v.at[pl.ds(0, tail)])
            pltpu.sync_copy(i1_hbm.at[pl.ds(base_w, tail)],
                            idx1_v.at[pl.ds(0, tail)])

        def drain(bufs, gabuf, semw):
            pltpu.make_async_copy(bufs, g_out.at[pl.ds(0, _C)], semw).wait()
            pltpu.make_async_copy(gabuf, ga_out.at[pl.ds(0, _C)], semw).wait()

        def issue(t, bufs, bufd, sem):
            off = t * _C
            return (
                pltpu.async_copy(tsrc_hbm.at[idx0_v.at[pl.ds(off, _C)]],
                                 bufs, sem),
                pltpu.async_copy(tdst_hbm.at[idx1_v.at[pl.ds(off, _C)]],
                                 bufd, sem),
            )

        def wait_issue(bufs, bufd, sem):
            pltpu.make_async_copy(tsrc_hbm.at[idx0_v.at[pl.ds(0, _C)]],
                                  bufs, sem).wait()
            pltpu.make_async_copy(tdst_hbm.at[idx1_v.at[pl.ds(0, _C)]],
                                  bufd, sem).wait()

        def ga_compute(t, gabuf):
            off = t * _C

            @pl.loop(0, _C, step=16)
            def _ga(k):
                v0 = idx0_v[pl.ds(off + k, 16)]
                v1 = idx1_v[pl.ds(off + k, 16)]
                gabuf[pl.ds(k, 16)] = (plsc.load_gather(pas_v, [v0]) +
                                       plsc.load_gather(pad_v, [v1]))

        def add_rows(bufs, bufd):
            @pl.loop(0, _C)
            def _row(r):
                for j in range(d2 // 16):
                    cut = pl.ds(j * 16, 16)
                    va = plsc.bitcast(bufs[r, cut], jnp.bfloat16)
                    vb = plsc.bitcast(bufd[r, cut], jnp.bfloat16)
                    bufs[r, cut] = plsc.bitcast(va + vb, jnp.int32)

        def writeback(t, bufs, gabuf, semw):
            base = base_w + t * _C
            pltpu.async_copy(bufs, g_out.at[pl.ds(base, _C)], semw)
            pltpu.async_copy(gabuf, ga_out.at[pl.ds(base, _C)], semw)

        @pl.loop(0, per // _C, step=2)
        def _pair(t0):
            t1 = t0 + 1

            @pl.when(t0 < nch)
            def _():
                @pl.when(t0 > 0)
                def _():
                    drain(bsa, gaa, semwa)
                issue(t0, bsa, bda, sema)

                @pl.when(t1 < nch)
                def _():
                    @pl.when(t0 > 0)
                    def _():
                        drain(bsb, gab, semwb)
                    issue(t1, bsb, bdb, semb)

                ga_compute(t0, gaa)
                wait_issue(bsa, bda, sema)
                add_rows(bsa, bda)
                writeback(t0, bsa, gaa, semwa)

                @pl.when(t1 < nch)
                def _():
                    ga_compute(t1, gab)
                    wait_issue(bsb, bdb, semb)
                    add_rows(bsb, bdb)
                    writeback(t1, bsb, gab, semwb)

        drain(bsa, gaa, semwa)

        @pl.when(nch >= 2)
        def _():
            drain(bsb, gab, semwb)

    return fn(tsrc, tdst, i0, i1, pas, pad)


def _make_edge1_body(has_prev):
    def body(*refs):
        if has_prev:
            (x_ref, g_ref, ga_ref, we_ref, wae_ref, ub_ref, ab_ref,
             w2_ref, lb_ref, _xsp_ref, _mrp_ref, mp_ref, zp_ref,
             xo_ref, mrow_ref, m_ref, z_ref) = refs
        else:
            (x_ref, g_ref, ga_ref, we_ref, wae_ref, ub_ref, ab_ref,
             w2_ref, lb_ref, xo_ref, mrow_ref, m_ref, z_ref) = refs
        pid = pl.program_id(0)
        d = x_ref.shape[1]
        x = x_ref[...]
        gi = g_ref[...]
        gu = lax.bitcast_convert_type(gi << 16, jnp.float32)
        gl = lax.bitcast_convert_type(gi & jnp.int32(-65536), jnp.float32)
        eu = jnp.dot(x.astype(jnp.bfloat16),
                     we_ref[...].astype(jnp.bfloat16),
                     preferred_element_type=jnp.float32)
        eu = eu + ub_ref[...]
        ea = jnp.dot(x, wae_ref[...], preferred_element_type=jnp.float32)
        pre = ea + ab_ref[...] + ga_ref[...]
        pre = jnp.where(pre >= 0, pre, 0.2 * pre)
        a = jax.nn.sigmoid(pre)
        upd = (gu + eu) * a
        ef = jnp.dot(upd.astype(jnp.bfloat16),
                     w2_ref[...].astype(jnp.bfloat16),
                     preferred_element_type=jnp.float32)
        s = ef + lb_ref[...] + gl
        s = jnp.where(s >= 0, s, 0.2 * s)

        @pl.when(pid == 0)
        def _():
            if has_prev:
                m_ref[...] = mp_ref[...]
                z_ref[...] = zp_ref[...]
            else:
                m_ref[...] = jnp.full(m_ref.shape, -jnp.inf, jnp.float32)
                z_ref[...] = jnp.zeros(z_ref.shape, jnp.float32)

        m_old = m_ref[...]
        m_new = jnp.maximum(m_old, jnp.max(s, axis=0, keepdims=True))
        expv = jnp.exp(s - m_new)
        z_new = z_ref[...] * jnp.exp(m_old - m_new) + jnp.sum(
            expv, axis=0, keepdims=True)
        m_ref[...] = m_new
        z_ref[...] = z_new
        xo_ref[...] = (upd * expv).astype(jnp.bfloat16)
        mrow_ref[...] = m_new.reshape(mrow_ref.shape)

    return body


def _edge_pass1(x, g, ga, we, wae, ub, ab, w2, lb, be, off, nb_total,
                prev=None):
    eh = g.shape[0]
    d = x.shape[1]
    e = x.shape[0]
    grid = (eh // be,)
    full = lambda shape: pl.BlockSpec(shape, lambda i: (0, 0))
    in_specs = [
        pl.BlockSpec((be, d), lambda i: (i + off, 0)),
        pl.BlockSpec((be, d), lambda i: (i, 0)),
        pl.BlockSpec((be, 1), lambda i: (i, 0)),
        full((d, d)), full((d, 1)), full((1, d)), full((1, 1)),
        full((d, d)), full((1, d)),
    ]
    operands = [x, g, ga, we, wae, ub, ab, w2, lb]
    aliases = {}
    if prev is not None:
        xsp, mrp, mp, zp = prev
        in_specs += [
            pl.BlockSpec((8, d), lambda i: (0, 0)),
            pl.BlockSpec((1, 1, d), lambda i: (0, 0, 0)),
            full((1, d)), full((1, d)),
        ]
        operands += [xsp, mrp, mp, zp]
        aliases = {9: 0, 10: 1}
    return pl.pallas_call(
        _make_edge1_body(prev is not None),
        grid=grid,
        in_specs=in_specs,
        out_specs=[
            pl.BlockSpec((be, d), lambda i: (i + off, 0)),
            pl.BlockSpec((1, 1, d), lambda i: (i + off, 0, 0)),
            pl.BlockSpec((1, d), lambda i: (0, 0)),
            pl.BlockSpec((1, d), lambda i: (0, 0)),
        ],
        out_shape=[
            jax.ShapeDtypeStruct((e, d), jnp.bfloat16),
            jax.ShapeDtypeStruct((nb_total, 1, d), jnp.float32),
            jax.ShapeDtypeStruct((1, d), jnp.float32),
            jax.ShapeDtypeStruct((1, d), jnp.float32),
        ],
        input_output_aliases=aliases,
        compiler_params=pltpu.CompilerParams(
            dimension_semantics=("arbitrary",)),
    )(*operands)


def _edge2_body(x_ref, mrow_ref, m_ref, z_ref, o_ref):
    scale = jnp.exp(mrow_ref[0] - m_ref[...]) / z_ref[...]
    o_ref[...] = x_ref[...].astype(jnp.float32) * scale


def _edge_pass2(xs, mrow, m, z, be):
    e, d = xs.shape
    grid = (e // be,)
    return pl.pallas_call(
        _edge2_body,
        grid=grid,
        in_specs=[
            pl.BlockSpec((be, d), lambda i: (i, 0)),
            pl.BlockSpec((1, 1, d), lambda i: (i, 0, 0)),
            pl.BlockSpec((1, d), lambda i: (0, 0)),
            pl.BlockSpec((1, d), lambda i: (0, 0)),
        ],
        out_specs=pl.BlockSpec((be, d), lambda i: (i, 0)),
        out_shape=jax.ShapeDtypeStruct((e, d), jnp.float32),
        compiler_params=pltpu.CompilerParams(
            dimension_semantics=("arbitrary",)),
    )(xs, mrow, m, z)


def kernel(edge_embeddings, edge_index, edge_attr, node_embeddings,
           num_nodes, attn_W, attn_b, update_W, update_b,
           edge_lin_W, edge_lin_b, node_lin_W, node_lin_b):
    del edge_attr, num_nodes
    e, d = edge_embeddings.shape
    n = node_embeddings.shape[0]

    i0 = edge_index[0].astype(jnp.int32)
    i1 = edge_index[1].astype(jnp.int32)

    ws, wd, we = update_W[:d], update_W[d:2 * d], update_W[2 * d:]
    wa2 = jnp.concatenate([attn_W[:d], attn_W[d:2 * d]], axis=1)
    wae = attn_W[2 * d:]

    bn = 2000 if n % 2000 == 0 else n
    tsrc, tdst, p2 = _node_precompute(
        node_embeddings, ws, wd, node_lin_W,
        node_lin_b.reshape(1, d), wa2, bn)

    pas, pad = p2[:, 0], p2[:, 1]
    be = 2000 if e % 2000 == 0 else e
    nb_total = e // be
    ub = update_b.reshape(1, d)
    ab = attn_b.reshape(1, 1)
    lb = edge_lin_b.reshape(1, d)

    half = e // 2
    if half % be == 0 and half % _C == 0:
        g0, ga0 = _sc_gather(tsrc, tdst, i0[:half], i1[:half], pas, pad)
        g1, ga1 = _sc_gather(tsrc, tdst, i0[half:], i1[half:], pas, pad)
        prev = _edge_pass1(edge_embeddings, g0, ga0.reshape(half, 1), we,
                           wae, ub, ab, edge_lin_W, lb, be, 0, nb_total)
        xs, mrow, m, z = _edge_pass1(
            edge_embeddings, g1, ga1.reshape(half, 1), we, wae, ub, ab,
            edge_lin_W, lb, be, half // be, nb_total, prev=prev)
    else:
        g, ga = _sc_gather(tsrc, tdst, i0, i1, pas, pad)
        xs, mrow, m, z = _edge_pass1(
            edge_embeddings, g, ga.reshape(e, 1), we, wae, ub, ab,
            edge_lin_W, lb, be, 0, nb_total)

    return _edge_pass2(xs, mrow, m, z, be)


# speedup vs baseline: 2.8863x; 1.0924x over previous
import dataclasses
import functools

import jax
import jax.numpy as jnp
from jax import lax
from jax.experimental import pallas as pl
from jax.experimental.pallas import tpu as pltpu
from jax.experimental.pallas import tpu_sc as plsc


def _pack16(u, l):
    ub = lax.bitcast_convert_type(u.astype(jnp.bfloat16).astype(jnp.float32),
                                  jnp.int32)
    lb = lax.bitcast_convert_type(l.astype(jnp.bfloat16).astype(jnp.float32),
                                  jnp.int32)
    return ((ub >> 16) & jnp.int32(0xFFFF)) | (lb & jnp.int32(-65536))


def _node_body(emb_ref, ws_ref, wd_ref, wl_ref, nb_ref, wa_ref,
               tsrc_ref, tdst_ref, p2_ref):
    emb = emb_ref[...]
    pl_feat = jnp.dot(emb, wl_ref[...], preferred_element_type=jnp.float32)
    pl_feat = pl_feat + nb_ref[...]
    ps = jnp.dot(emb, ws_ref[...], preferred_element_type=jnp.float32)
    pd = jnp.dot(emb, wd_ref[...], preferred_element_type=jnp.float32)
    tsrc_ref[...] = _pack16(ps, pl_feat)
    tdst_ref[...] = _pack16(pd, pl_feat)
    p2_ref[...] = jnp.dot(emb, wa_ref[...], preferred_element_type=jnp.float32)


def _node_precompute(emb, ws, wd, wl, nb, wa, bn):
    n, d = emb.shape
    grid = (n // bn,)
    full = lambda shape: pl.BlockSpec(shape, lambda i: (0, 0))
    return pl.pallas_call(
        _node_body,
        grid=grid,
        in_specs=[
            pl.BlockSpec((bn, d), lambda i: (i, 0)),
            full((d, d)), full((d, d)), full((d, d)),
            full((1, d)), full((d, 2)),
        ],
        out_specs=[
            pl.BlockSpec((bn, d), lambda i: (i, 0)),
            pl.BlockSpec((bn, d), lambda i: (i, 0)),
            pl.BlockSpec((bn, 2), lambda i: (i, 0)),
        ],
        out_shape=[
            jax.ShapeDtypeStruct((n, d), jnp.int32),
            jax.ShapeDtypeStruct((n, d), jnp.int32),
            jax.ShapeDtypeStruct((n, 2), jnp.float32),
        ],
        compiler_params=pltpu.CompilerParams(
            dimension_semantics=("arbitrary",)),
    )(emb, ws, wd, wl, nb, wa)


_C = 32


def _sc_gather(tsrc, tdst, i0, i1, pas, pad):
    e = i0.shape[0]
    n, d2 = tsrc.shape
    mesh = plsc.VectorSubcoreMesh(core_axis_name="c", subcore_axis_name="s")
    nw = 32
    chunks = e // _C

    cp = pltpu.CompilerParams()
    if "needs_layout_passes" in pltpu.CompilerParams.__dataclass_fields__:
        cp = dataclasses.replace(cp, needs_layout_passes=False)

    per = -(-(e // nw) // _C) * _C
    tail = e - (nw - 1) * per
    assert tail > 0 and tail % _C == 0 and per % 8 == 0

    buf_t = pltpu.VMEM((_C, d2), jnp.int32)
    ga_t = pltpu.VMEM((_C,), jnp.float32)

    @functools.partial(
        pl.kernel, mesh=mesh,
        compiler_params=cp,
        out_type=(jax.ShapeDtypeStruct((e, d2), jnp.int32),
                  jax.ShapeDtypeStruct((e,), jnp.float32)),
        scratch_types=[
            pltpu.VMEM((per,), jnp.int32), pltpu.VMEM((per,), jnp.int32),
            buf_t, buf_t, buf_t, buf_t,
            ga_t, ga_t,
            pltpu.VMEM((n,), jnp.float32), pltpu.VMEM((n,), jnp.float32),
            pltpu.SemaphoreType.DMA, pltpu.SemaphoreType.DMA,
            pltpu.SemaphoreType.DMA, pltpu.SemaphoreType.DMA,
        ],
    )
    def fn(tsrc_hbm, tdst_hbm, i0_hbm, i1_hbm, pas_hbm, pad_hbm,
           g_out, ga_out,
           idx0_v, idx1_v, bsa, bda, bsb, bdb, gaa, gab,
           pas_v, pad_v, sema, semb, semwa, semwb):
        cid = lax.axis_index("c")
        sid = lax.axis_index("s")
        wid = sid * 2 + cid
        base_w = wid * per
        nch = jnp.where(wid == nw - 1, tail // _C, per // _C)
        pltpu.sync_copy(pas_hbm, pas_v)
        pltpu.sync_copy(pad_hbm, pad_v)

        @pl.when(wid < nw - 1)
        def _():
            pltpu.sync_copy(i0_hbm.at[pl.ds(base_w, per)], idx0_v)
            pltpu.sync_copy(i1_hbm.at[pl.ds(base_w, per)], idx1_v)

        @pl.when(wid == nw - 1)
        def _():
            pltpu.sync_copy(i0_hbm.at[pl.ds(base_w, tail)],
                            idx0_v.at[pl.ds(0, tail)])
            pltpu.sync_copy(i1_hbm.at[pl.ds(base_w, tail)],
                            idx1_v.at[pl.ds(0, tail)])

        def drain(bufs, gabuf, semw):
            pltpu.make_async_copy(bufs, g_out.at[pl.ds(0, _C)], semw).wait()
            pltpu.make_async_copy(gabuf, ga_out.at[pl.ds(0, _C)], semw).wait()

        def issue(t, bufs, bufd, sem):
            off = t * _C
            return (
                pltpu.async_copy(tsrc_hbm.at[idx0_v.at[pl.ds(off, _C)]],
                                 bufs, sem),
                pltpu.async_copy(tdst_hbm.at[idx1_v.at[pl.ds(off, _C)]],
                                 bufd, sem),
            )

        def wait_issue(bufs, bufd, sem):
            pltpu.make_async_copy(tsrc_hbm.at[idx0_v.at[pl.ds(0, _C)]],
                                  bufs, sem).wait()
            pltpu.make_async_copy(tdst_hbm.at[idx1_v.at[pl.ds(0, _C)]],
                                  bufd, sem).wait()

        def ga_compute(t, gabuf):
            off = t * _C

            @pl.loop(0, _C, step=16)
            def _ga(k):
                v0 = idx0_v[pl.ds(off + k, 16)]
                v1 = idx1_v[pl.ds(off + k, 16)]
                gabuf[pl.ds(k, 16)] = (plsc.load_gather(pas_v, [v0]) +
                                       plsc.load_gather(pad_v, [v1]))

        def add_rows(bufs, bufd):
            @pl.loop(0, _C)
            def _row(r):
                for j in range(d2 // 16):
                    cut = pl.ds(j * 16, 16)
                    va = plsc.bitcast(bufs[r, cut], jnp.bfloat16)
                    vb = plsc.bitcast(bufd[r, cut], jnp.bfloat16)
                    bufs[r, cut] = plsc.bitcast(va + vb, jnp.int32)

        def writeback(t, bufs, gabuf, semw):
            base = base_w + t * _C
            pltpu.async_copy(bufs, g_out.at[pl.ds(base, _C)], semw)
            pltpu.async_copy(gabuf, ga_out.at[pl.ds(base, _C)], semw)

        @pl.loop(0, per // _C, step=2)
        def _pair(t0):
            t1 = t0 + 1

            @pl.when(t0 < nch)
            def _():
                @pl.when(t0 > 0)
                def _():
                    drain(bsa, gaa, semwa)
                issue(t0, bsa, bda, sema)

                @pl.when(t1 < nch)
                def _():
                    @pl.when(t0 > 0)
                    def _():
                        drain(bsb, gab, semwb)
                    issue(t1, bsb, bdb, semb)

                ga_compute(t0, gaa)
                wait_issue(bsa, bda, sema)
                add_rows(bsa, bda)
                writeback(t0, bsa, gaa, semwa)

                @pl.when(t1 < nch)
                def _():
                    ga_compute(t1, gab)
                    wait_issue(bsb, bdb, semb)
                    add_rows(bsb, bdb)
                    writeback(t1, bsb, gab, semwb)

        drain(bsa, gaa, semwa)

        @pl.when(nch >= 2)
        def _():
            drain(bsb, gab, semwb)

    return fn(tsrc, tdst, i0, i1, pas, pad)


def _make_edge1_body(has_prev):
    def body(*refs):
        if has_prev:
            (x_ref, g_ref, ga_ref, we_ref, wae_ref, ub_ref, ab_ref,
             w2_ref, lb_ref, _xsp_ref, _mrp_ref, mp_ref, zp_ref,
             xo_ref, mrow_ref, m_ref, z_ref) = refs
        else:
            (x_ref, g_ref, ga_ref, we_ref, wae_ref, ub_ref, ab_ref,
             w2_ref, lb_ref, xo_ref, mrow_ref, m_ref, z_ref) = refs
        pid = pl.program_id(0)
        d = x_ref.shape[1]
        x = x_ref[...]
        gi = g_ref[...]
        gu = lax.bitcast_convert_type(gi << 16, jnp.float32)
        gl = lax.bitcast_convert_type(gi & jnp.int32(-65536), jnp.float32)
        eu = jnp.dot(x.astype(jnp.bfloat16),
                     we_ref[...].astype(jnp.bfloat16),
                     preferred_element_type=jnp.float32)
        eu = eu + ub_ref[...]
        ea = jnp.dot(x, wae_ref[...], preferred_element_type=jnp.float32)
        pre = ea + ab_ref[...] + ga_ref[...]
        pre = jnp.where(pre >= 0, pre, 0.2 * pre)
        a = jax.nn.sigmoid(pre)
        upd = (gu + eu) * a
        ef = jnp.dot(upd.astype(jnp.bfloat16),
                     w2_ref[...].astype(jnp.bfloat16),
                     preferred_element_type=jnp.float32)
        s = ef + lb_ref[...] + gl
        s = jnp.where(s >= 0, s, 0.2 * s)

        @pl.when(pid == 0)
        def _():
            if has_prev:
                m_ref[...] = mp_ref[...]
                z_ref[...] = zp_ref[...]
            else:
                m_ref[...] = jnp.full(m_ref.shape, -jnp.inf, jnp.float32)
                z_ref[...] = jnp.zeros(z_ref.shape, jnp.float32)

        m_old = m_ref[...]
        m_new = jnp.maximum(m_old, jnp.max(s, axis=0, keepdims=True))
        expv = jnp.exp(s - m_new)
        z_new = z_ref[...] * jnp.exp(m_old - m_new) + jnp.sum(
            expv, axis=0, keepdims=True)
        m_ref[...] = m_new
        z_ref[...] = z_new
        xo_ref[...] = (upd * expv).astype(jnp.bfloat16)
        mrow_ref[...] = m_new.reshape(mrow_ref.shape)

    return body


def _edge_pass1(x, g, ga, we, wae, ub, ab, w2, lb, be, off, nb_total,
                prev=None):
    eh = g.shape[0]
    d = x.shape[1]
    e = x.shape[0]
    grid = (eh // be,)
    full = lambda shape: pl.BlockSpec(shape, lambda i: (0, 0))
    in_specs = [
        pl.BlockSpec((be, d), lambda i: (i + off, 0)),
        pl.BlockSpec((be, d), lambda i: (i, 0)),
        pl.BlockSpec((be, 1), lambda i: (i, 0)),
        full((d, d)), full((d, 1)), full((1, d)), full((1, 1)),
        full((d, d)), full((1, d)),
    ]
    operands = [x, g, ga, we, wae, ub, ab, w2, lb]
    aliases = {}
    if prev is not None:
        xsp, mrp, mp, zp = prev
        in_specs += [
            pl.BlockSpec((8, d), lambda i: (0, 0)),
            pl.BlockSpec((1, 1, d), lambda i: (0, 0, 0)),
            full((1, d)), full((1, d)),
        ]
        operands += [xsp, mrp, mp, zp]
        aliases = {9: 0, 10: 1}
    return pl.pallas_call(
        _make_edge1_body(prev is not None),
        grid=grid,
        in_specs=in_specs,
        out_specs=[
            pl.BlockSpec((be, d), lambda i: (i + off, 0)),
            pl.BlockSpec((1, 1, d), lambda i: (i + off, 0, 0)),
            pl.BlockSpec((1, d), lambda i: (0, 0)),
            pl.BlockSpec((1, d), lambda i: (0, 0)),
        ],
        out_shape=[
            jax.ShapeDtypeStruct((e, d), jnp.bfloat16),
            jax.ShapeDtypeStruct((nb_total, 1, d), jnp.float32),
            jax.ShapeDtypeStruct((1, d), jnp.float32),
            jax.ShapeDtypeStruct((1, d), jnp.float32),
        ],
        input_output_aliases=aliases,
        compiler_params=pltpu.CompilerParams(
            dimension_semantics=("arbitrary",)),
    )(*operands)


def _edge2_body(x_ref, mrow_ref, m_ref, z_ref, o_ref):
    scale = jnp.exp(mrow_ref[0] - m_ref[...]) / z_ref[...]
    o_ref[...] = x_ref[...].astype(jnp.float32) * scale


def _edge_pass2(xs, mrow, m, z, be):
    e, d = xs.shape
    grid = (e // be,)
    return pl.pallas_call(
        _edge2_body,
        grid=grid,
        in_specs=[
            pl.BlockSpec((be, d), lambda i: (i, 0)),
            pl.BlockSpec((1, 1, d), lambda i: (i, 0, 0)),
            pl.BlockSpec((1, d), lambda i: (0, 0)),
            pl.BlockSpec((1, d), lambda i: (0, 0)),
        ],
        out_specs=pl.BlockSpec((be, d), lambda i: (i, 0)),
        out_shape=jax.ShapeDtypeStruct((e, d), jnp.float32),
        compiler_params=pltpu.CompilerParams(
            dimension_semantics=("arbitrary",)),
    )(xs, mrow, m, z)


def kernel(edge_embeddings, edge_index, edge_attr, node_embeddings,
           num_nodes, attn_W, attn_b, update_W, update_b,
           edge_lin_W, edge_lin_b, node_lin_W, node_lin_b):
    del edge_attr, num_nodes
    e, d = edge_embeddings.shape
    n = node_embeddings.shape[0]

    i0 = edge_index[0].astype(jnp.int32)
    i1 = edge_index[1].astype(jnp.int32)

    ws, wd, we = update_W[:d], update_W[d:2 * d], update_W[2 * d:]
    wa2 = jnp.concatenate([attn_W[:d], attn_W[d:2 * d]], axis=1)
    wae = attn_W[2 * d:]

    bn = 2000 if n % 2000 == 0 else n
    tsrc, tdst, p2 = _node_precompute(
        node_embeddings, ws, wd, node_lin_W,
        node_lin_b.reshape(1, d), wa2, bn)

    pas, pad = p2[:, 0], p2[:, 1]
    be = 2000 if e % 2000 == 0 else e
    nb_total = e // be
    ub = update_b.reshape(1, d)
    ab = attn_b.reshape(1, 1)
    lb = edge_lin_b.reshape(1, d)

    half = e // 2
    if half % be == 0 and half % _C == 0:
        g0, ga0 = _sc_gather(tsrc, tdst, i0[:half], i1[:half], pas, pad)
        g1, ga1 = _sc_gather(tsrc, tdst, i0[half:], i1[half:], pas, pad)
        xs0, mrow0, m0, z0 = _edge_pass1(
            edge_embeddings, g0, ga0.reshape(half, 1), we,
            wae, ub, ab, edge_lin_W, lb, be, 0, nb_total)
        g1, ga1, m0, z0 = lax.optimization_barrier((g1, ga1, m0, z0))
        xs, mrow, m, z = _edge_pass1(
            edge_embeddings, g1, ga1.reshape(half, 1), we, wae, ub, ab,
            edge_lin_W, lb, be, half // be, nb_total,
            prev=(xs0, mrow0, m0, z0))
    else:
        g, ga = _sc_gather(tsrc, tdst, i0, i1, pas, pad)
        xs, mrow, m, z = _edge_pass1(
            edge_embeddings, g, ga.reshape(e, 1), we, wae, ub, ab,
            edge_lin_W, lb, be, 0, nb_total)

    return _edge_pass2(xs, mrow, m, z, be)
